# baseline (speedup 1.0000x reference)
import functools

import jax
import jax.numpy as jnp
from jax import lax
from jax.experimental import pallas as pl
from jax.experimental.pallas import tpu as pltpu
from jax.experimental.pallas import tpu_sc as plsc

N = 10000
E = 160000
IN_FEAT = 128
HID = 64
HEADS = 4
NEG_SLOPE = 0.2

NC = 2
NS = 16
L = 16

NR = 10240
DUMP = N
RPT = NR // NS

ET1 = E // NS
EC = 64
NFULL = ET1 // EC
TAILE = ET1 - NFULL * EC
C1 = 128
NCH1 = 79
TAIL1 = ET1 - (NCH1 - 1) * C1

NH = N // NC
NR2 = 5120
RPT2 = NR2 // NS


def _mesh():
    return plsc.VectorSubcoreMesh(core_axis_name="c", subcore_axis_name="s",
                                  num_cores=NC, num_subcores=NS)


def _k1_body(x_ref, ws_ref, wd_ref, fs_ref, fd_ref):
    x = x_ref[...]
    fs_ref[...] = jnp.dot(x, ws_ref[...], preferred_element_type=jnp.float32)
    fd_ref[...] = jnp.dot(x, wd_ref[...], preferred_element_type=jnp.float32)


def _k1(x, W_src1, W_dst1):
    nb = 10
    rb = N // nb
    return pl.pallas_call(
        _k1_body,
        grid=(NC, nb),
        in_specs=[
            pl.BlockSpec((rb, IN_FEAT), lambda c, j: (j, 0)),
            pl.BlockSpec((IN_FEAT, 128), lambda c, j: (0, c)),
            pl.BlockSpec((IN_FEAT, 128), lambda c, j: (0, c)),
        ],
        out_specs=[
            pl.BlockSpec((rb, 128), lambda c, j: (c * nb + j, 0)),
            pl.BlockSpec((rb, 128), lambda c, j: (c * nb + j, 0)),
        ],
        out_shape=[
            jax.ShapeDtypeStruct((NC * N, 128), jnp.float32),
            jax.ShapeDtypeStruct((NC * N, 128), jnp.float32),
        ],
    )(x, W_src1, W_dst1)


def _k2_body(fs_hbm, fd_hbm, src_hbm, dst_hbm, attn_hbm,
             out_hbm, outden_hbm,
             src_ch, dstoff_ch, dstrow_ch, fs_rows, fd_rows, attn_v,
             den0_v, den1_v, acc_s, sem1, sem2):
    c = lax.axis_index("c")
    s = lax.axis_index("s")
    zf = jnp.zeros((L,), jnp.float32)
    lane = lax.iota(jnp.int32, L)

    def _zrow(r, carry):
        for k in range(128 // L):
            fs_rows[r, pl.ds(k * L, L)] = zf
        return carry
    lax.fori_loop(0, EC, _zrow, 0)
    base = s * RPT
    for blk in range(RPT // EC):
        pltpu.sync_copy(fs_rows, acc_s.at[pl.ds(base + blk * EC, EC)])

    def _zden(i, carry):
        den0_v[pl.ds(i * L, L)] = zf
        den1_v[pl.ds(i * L, L)] = zf
        return carry
    lax.fori_loop(0, NR // L, _zden, 0)

    pltpu.sync_copy(attn_hbm.at[c], attn_v)
    a_vecs = [attn_v[pl.ds(k * L, L)] for k in range(8)]

    coff = jnp.full((L,), c * N, jnp.int32)
    ebase = s * ET1

    plsc.subcore_barrier()

    def _do_chunk():
        cp1 = pltpu.async_copy(fs_hbm.at[src_ch], fs_rows, sem1)
        cp2 = pltpu.async_copy(fd_hbm.at[dstoff_ch], fd_rows, sem2)
        cp1.wait()
        cp2.wait()

        def _group(g, carry):
            dvec = dstrow_ch[pl.ds(g * L, L)]
            for e16 in range(L):
                e = g * L + e16
                fsv = [fs_rows[e, pl.ds(k * L, L)] for k in range(8)]
                fdv = [fd_rows[e, pl.ds(k * L, L)] for k in range(8)]
                av = []
                for k in range(8):
                    ev = fsv[k] + fdv[k]
                    ev = jnp.where(ev > 0, ev, NEG_SLOPE * ev)
                    av.append(ev * a_vecs[k])
                s0 = jnp.sum((av[0] + av[1]) + (av[2] + av[3]))
                s1 = jnp.sum((av[4] + av[5]) + (av[6] + av[7]))
                ex0 = jnp.exp(jnp.full((L,), s0, jnp.float32))
                ex1 = jnp.exp(jnp.full((L,), s1, jnp.float32))
                for k in range(4):
                    fs_rows[e, pl.ds(k * L, L)] = fsv[k] * ex0
                for k in range(4, 8):
                    fs_rows[e, pl.ds(k * L, L)] = fsv[k] * ex1
                plsc.addupdate_scatter(den0_v, [dvec], ex0, mask=lane == e16)
                plsc.addupdate_scatter(den1_v, [dvec], ex1, mask=lane == e16)
            return carry
        lax.fori_loop(0, EC // L, _group, 0)
        pltpu.sync_copy(fs_rows, acc_s.at[dstrow_ch], add=True)

    def _offset_indices():
        for g in range(EC // L):
            src_ch[pl.ds(g * L, L)] = src_ch[pl.ds(g * L, L)] + coff
            dstoff_ch[pl.ds(g * L, L)] = dstrow_ch[pl.ds(g * L, L)] + coff

    def _chunk(j, carry):
        eb = ebase + j * EC
        pltpu.sync_copy(src_hbm.at[pl.ds(eb, EC)], src_ch)
        pltpu.sync_copy(dst_hbm.at[pl.ds(eb, EC)], dstrow_ch)
        _offset_indices()
        _do_chunk()
        return carry
    lax.fori_loop(0, NFULL, _chunk, 0)

    pltpu.sync_copy(src_hbm.at[pl.ds(ebase + NFULL * EC, TAILE)],
                    src_ch.at[pl.ds(0, TAILE)])
    pltpu.sync_copy(dst_hbm.at[pl.ds(ebase + NFULL * EC, TAILE)],
                    dstrow_ch.at[pl.ds(0, TAILE)])
    for t in range(TAILE // L, EC // L):
        src_ch[pl.ds(t * L, L)] = jnp.full((L,), 0, jnp.int32)
        dstrow_ch[pl.ds(t * L, L)] = jnp.full((L,), DUMP, jnp.int32)
    _offset_indices()
    _do_chunk()

    pltpu.sync_copy(den0_v, outden_hbm.at[c, s, 0])
    pltpu.sync_copy(den1_v, outden_hbm.at[c, s, 1])

    plsc.subcore_barrier()
    pltpu.sync_copy(acc_s.at[pl.ds(base, RPT)], out_hbm.at[c, pl.ds(base, RPT)])


def _k2(fs_t, fd_t, src, dst, attn_t):
    f = functools.partial(
        pl.kernel,
        out_type=(
            jax.ShapeDtypeStruct((NC, NR, 128), jnp.float32),
            jax.ShapeDtypeStruct((NC, NS, 2, NR), jnp.float32),
        ),
        mesh=_mesh(),
        compiler_params=pltpu.CompilerParams(needs_layout_passes=False),
        scratch_types=[
            pltpu.VMEM((EC,), jnp.int32),
            pltpu.VMEM((EC,), jnp.int32),
            pltpu.VMEM((EC,), jnp.int32),
            pltpu.VMEM((EC, 128), jnp.float32),
            pltpu.VMEM((EC, 128), jnp.float32),
            pltpu.VMEM((128,), jnp.float32),
            pltpu.VMEM((NR,), jnp.float32),
            pltpu.VMEM((NR,), jnp.float32),
            pltpu.VMEM_SHARED((NR, 128), jnp.float32),
            pltpu.SemaphoreType.DMA,
            pltpu.SemaphoreType.DMA,
        ],
    )(_k2_body)
    return f(fs_t, fd_t, src, dst, attn_t)


def _k3_body(acc_ref, den_ref, b_ref, ws_ref, wd_ref, ft_ref):
    a = acc_ref[...]
    d = den_ref[...]
    parts = []
    for c in range(NC):
        for k in range(2):
            num = a[c, :, k * 64:(k + 1) * 64]
            den = jnp.sum(d[:, (c * 2 + k) * NS:(c * 2 + k + 1) * NS],
                          axis=1, keepdims=True)
            parts.append(num / jnp.maximum(den, 1e-16))
    h = jnp.concatenate(parts, axis=1) + b_ref[...]
    h = jnp.where(h > 0, h, jnp.exp(h) - 1.0)
    ft_ref[...] = jnp.concatenate(
        [jnp.dot(h, ws_ref[...], preferred_element_type=jnp.float32),
         jnp.dot(h, wd_ref[...], preferred_element_type=jnp.float32)], axis=1)


def _k3(acc1, den1, bias1, W_src2, W_dst2):
    nb = 10
    rb = N // nb
    return pl.pallas_call(
        _k3_body,
        grid=(nb,),
        in_specs=[
            pl.BlockSpec((NC, rb, 128), lambda j: (0, j, 0)),
            pl.BlockSpec((rb, NC * 2 * NS), lambda j: (j, 0)),
            pl.BlockSpec((1, HEADS * HID), lambda j: (0, 0)),
            pl.BlockSpec((HEADS * HID, HID), lambda j: (0, 0)),
            pl.BlockSpec((HEADS * HID, HID), lambda j: (0, 0)),
        ],
        out_specs=[
            pl.BlockSpec((rb, 2 * HID), lambda j: (j, 0)),
        ],
        out_shape=[
            jax.ShapeDtypeStruct((N, 2 * HID), jnp.float32),
        ],
    )(acc1, den1, bias1, W_src2, W_dst2)


def _k4_body(ft_hbm, src_hbm, dst_hbm, attn_hbm, out_hbm,
             src_v, dst_v, dst2d, fs_rows, fd_rows, attn_v,
             acc_s, sem1, sem2):
    c = lax.axis_index("c")
    s = lax.axis_index("s")
    zf = jnp.zeros((L,), jnp.float32)
    lane = lax.iota(jnp.int32, L)

    def _zrow(r, carry):
        for k in range(128 // L):
            fs_rows[r, pl.ds(k * L, L)] = zf
        return carry
    lax.fori_loop(0, C1, _zrow, 0)
    base = s * RPT2
    for blk in range(RPT2 // C1):
        pltpu.sync_copy(fs_rows, acc_s.at[pl.ds(base + blk * C1, C1)])
    rem = RPT2 - (RPT2 // C1) * C1
    pltpu.sync_copy(fs_rows.at[pl.ds(0, rem)],
                    acc_s.at[pl.ds(base + (RPT2 // C1) * C1, rem)])

    ebase = s * ET1
    pltpu.sync_copy(src_hbm.at[pl.ds(ebase, ET1)], src_v.at[pl.ds(0, ET1)])
    pltpu.sync_copy(dst_hbm.at[pl.ds(ebase, ET1)], dst_v.at[pl.ds(0, ET1)])
    for t in range(7):
        src_v[pl.ds(ET1 + t * L, L)] = jnp.full((L,), 0, jnp.int32)
        dst_v[pl.ds(ET1 + t * L, L)] = jnp.full((L,), 0, jnp.int32)

    for j in range(NCH1 - 1):
        pltpu.sync_copy(dst_hbm.at[pl.ds(ebase + j * C1, C1)], dst2d.at[j])
    pltpu.sync_copy(dst_hbm.at[pl.ds(ebase + (NCH1 - 1) * C1, TAIL1)],
                    dst2d.at[NCH1 - 1, pl.ds(0, TAIL1)])
    for t in range(TAIL1 // L, C1 // L):
        dst2d[NCH1 - 1, pl.ds(t * L, L)] = jnp.full((L,), N, jnp.int32)

    cnh = jnp.full((L,), c * NH, jnp.int32)
    dump = jnp.full((L,), NH, jnp.int32)

    def _loc(j, carry):
        for g in range(C1 // L):
            v = dst2d[j, pl.ds(g * L, L)] - cnh
            inr = (v >= 0) & (v < NH)
            dst2d[j, pl.ds(g * L, L)] = jnp.where(inr, v, dump)
        return carry
    lax.fori_loop(0, NCH1, _loc, 0)

    pltpu.sync_copy(attn_hbm.at[0], attn_v)
    a_vecs = [attn_v[pl.ds(k * L, L)] for k in range(4)]

    plsc.subcore_barrier()

    def _chunk(j, carry):
        jb = j * C1
        cp1 = pltpu.async_copy(ft_hbm.at[src_v.at[pl.ds(jb, C1)]], fs_rows, sem1)
        cp2 = pltpu.async_copy(ft_hbm.at[dst_v.at[pl.ds(jb, C1)]], fd_rows, sem2)
        cp1.wait()
        cp2.wait()

        def _edge(e, carry2):
            fsv = [fs_rows[e, pl.ds(k * L, L)] for k in range(4)]
            fdv = [fd_rows[e, pl.ds((4 + k) * L, L)] for k in range(4)]
            av = []
            for k in range(4):
                ev = fsv[k] + fdv[k]
                ev = jnp.where(ev > 0, ev, NEG_SLOPE * ev)
                av.append(ev * a_vecs[k])
            s0 = jnp.sum((av[0] + av[1]) + (av[2] + av[3]))
            ex0 = jnp.exp(jnp.full((L,), s0, jnp.float32))
            for k in range(4):
                fs_rows[e, pl.ds(k * L, L)] = fsv[k] * ex0
            fs_rows[e, pl.ds(64, L)] = jnp.where(lane == 0, ex0, zf)
            for k in range(5, 8):
                fs_rows[e, pl.ds(k * L, L)] = zf
            return carry2
        lax.fori_loop(0, C1, _edge, 0)
        pltpu.sync_copy(fs_rows, acc_s.at[dst2d.at[j]], add=True)
        return carry
    lax.fori_loop(0, NCH1, _chunk, 0)

    plsc.subcore_barrier()
    pltpu.sync_copy(acc_s.at[pl.ds(base, RPT2)], out_hbm.at[c, pl.ds(base, RPT2)])


def _k4(ft2, src, dst, attn2):
    f = functools.partial(
        pl.kernel,
        out_type=jax.ShapeDtypeStruct((NC, NR2, 128), jnp.float32),
        mesh=_mesh(),
        compiler_params=pltpu.CompilerParams(needs_layout_passes=False),
        scratch_types=[
            pltpu.VMEM((NCH1 * C1,), jnp.int32),
            pltpu.VMEM((NCH1 * C1,), jnp.int32),
            pltpu.VMEM((NCH1, C1), jnp.int32),
            pltpu.VMEM((C1, 128), jnp.float32),
            pltpu.VMEM((C1, 128), jnp.float32),
            pltpu.VMEM((HID,), jnp.float32),
            pltpu.VMEM_SHARED((NR2, 128), jnp.float32),
            pltpu.SemaphoreType.DMA,
            pltpu.SemaphoreType.DMA,
        ],
    )(_k4_body)
    return f(ft2, src, dst, attn2)


def _k5_body(acc_ref, b2_ref,
             wih0, whh0, bih0, bhh0, wih1, whh1, bih1, bhh1,
             wih2, whh2, bih2, bhh2, out_ref):
    a = acc_ref[...]
    num = jnp.concatenate([a[0, :NH, :HID], a[1, :NH, :HID]], axis=0)
    den = jnp.concatenate(
        [a[0, :NH, HID:HID + 1], a[1, :NH, HID:HID + 1]], axis=0)
    h = num / jnp.maximum(den, 1e-16) + b2_ref[...]
    feat = jnp.where(h > 0, h, jnp.exp(h) - 1.0)

    wihs = (wih0[...], wih1[...], wih2[...])
    whhs = (whh0[...], whh1[...], whh2[...])
    bihs = (bih0[...], bih1[...], bih2[...])
    bhhs = (bhh0[...], bhh1[...], bhh2[...])

    hs = [jnp.zeros((1, HID), jnp.float32) for _ in range(3)]
    cs = [jnp.zeros((1, HID), jnp.float32) for _ in range(3)]
    q_star = jnp.zeros((1, 2 * HID), jnp.float32)

    def dotT(u, w):
        return lax.dot_general(u, w, (((1,), (1,)), ((), ())),
                               preferred_element_type=jnp.float32)

    for _ in range(3):
        inp = q_star
        for l in range(3):
            gates = dotT(inp, wihs[l]) + bihs[l] + dotT(hs[l], whhs[l]) + bhhs[l]
            gi = gates[:, 0:HID]
            gf = gates[:, HID:2 * HID]
            gg = gates[:, 2 * HID:3 * HID]
            go = gates[:, 3 * HID:4 * HID]
            cnew = jax.nn.sigmoid(gf) * cs[l] + jax.nn.sigmoid(gi) * jnp.tanh(gg)
            hnew = jax.nn.sigmoid(go) * jnp.tanh(cnew)
            hs[l] = hnew
            cs[l] = cnew
            inp = hnew
        q = inp
        e = dotT(feat, q)
        m = jnp.max(e)
        z = jnp.exp(e - m)
        ssum = jnp.sum(z)
        r = lax.dot_general(z, feat, (((0,), (0,)), ((), ())),
                            preferred_element_type=jnp.float32) / ssum
        q_star = jnp.concatenate([q, r], axis=1)
    out_ref[...] = q_star


def _k5(acc2, bias2, lstm):
    return pl.pallas_call(
        _k5_body,
        out_shape=jax.ShapeDtypeStruct((1, 2 * HID), jnp.float32),
    )(acc2, bias2, *lstm)


def kernel(x, edge_index, W_src1, W_dst1, attn1, bias1, W_src2, W_dst2,
           attn2, bias2, W_ih0, W_hh0, b_ih0, b_hh0, W_ih1, W_hh1, b_ih1,
           b_hh1, W_ih2, W_hh2, b_ih2, b_hh2):
    src = edge_index[0]
    dst = edge_index[1]
    attn1_t = attn1.reshape(NC, 128)
    bias1_r = bias1.reshape(1, HEADS * HID)
    bias2_r = bias2.reshape(1, HID)

    fs_t, fd_t = _k1(x, W_src1, W_dst1)
    acc1, den1 = _k2(fs_t, fd_t, src, dst, attn1_t)
    den1_t = den1.transpose(3, 0, 2, 1).reshape(NR, NC * 2 * NS)
    ft2, = _k3(acc1, den1_t, bias1_r, W_src2, W_dst2)
    acc2 = _k4(ft2, src, dst, attn2)
    lstm = (W_ih0, W_hh0, b_ih0.reshape(1, -1), b_hh0.reshape(1, -1),
            W_ih1, W_hh1, b_ih1.reshape(1, -1), b_hh1.reshape(1, -1),
            W_ih2, W_hh2, b_ih2.reshape(1, -1), b_hh2.reshape(1, -1))
    return _k5(acc2, bias2_r, lstm)


# speedup vs baseline: 1.3018x; 1.3018x over previous
import functools

import jax
import jax.numpy as jnp
from jax import lax
from jax.experimental import pallas as pl
from jax.experimental.pallas import tpu as pltpu
from jax.experimental.pallas import tpu_sc as plsc

N = 10000
E = 160000
IN_FEAT = 128
HID = 64
HEADS = 4
NEG_SLOPE = 0.2

NC = 2
NS = 16
L = 16

NR = 10240
DUMP = N
RPT = NR // NS

ET1 = E // NS
EC = 64
NFULL = ET1 // EC
TAILE = ET1 - NFULL * EC
C1 = 128
NCH1 = 79
TAIL1 = ET1 - (NCH1 - 1) * C1

ET2 = E // (NC * NS)
NF2 = ET2 // EC
TAIL2 = ET2 - NF2 * EC


def _mesh():
    return plsc.VectorSubcoreMesh(core_axis_name="c", subcore_axis_name="s",
                                  num_cores=NC, num_subcores=NS)


def _k1_body(x_ref, ws_ref, wd_ref, fs_ref, fd_ref):
    x = x_ref[...]
    fs_ref[...] = jnp.dot(x, ws_ref[...], preferred_element_type=jnp.float32)
    fd_ref[...] = jnp.dot(x, wd_ref[...], preferred_element_type=jnp.float32)


def _k1(x, W_src1, W_dst1):
    nb = 10
    rb = N // nb
    return pl.pallas_call(
        _k1_body,
        grid=(NC, nb),
        in_specs=[
            pl.BlockSpec((rb, IN_FEAT), lambda c, j: (j, 0)),
            pl.BlockSpec((IN_FEAT, 128), lambda c, j: (0, c)),
            pl.BlockSpec((IN_FEAT, 128), lambda c, j: (0, c)),
        ],
        out_specs=[
            pl.BlockSpec((rb, 128), lambda c, j: (c * nb + j, 0)),
            pl.BlockSpec((rb, 128), lambda c, j: (c * nb + j, 0)),
        ],
        out_shape=[
            jax.ShapeDtypeStruct((NC * N, 128), jnp.float32),
            jax.ShapeDtypeStruct((NC * N, 128), jnp.float32),
        ],
    )(x, W_src1, W_dst1)


def _k2_body(fs_hbm, fd_hbm, src_hbm, dst_hbm, attn_hbm,
             out_hbm, outden_hbm,
             src_ch, dstoff_ch, dstrow_ch, fs_rows, fd_rows, attn_v,
             den0_v, den1_v, acc_s, sem1, sem2):
    c = lax.axis_index("c")
    s = lax.axis_index("s")
    zf = jnp.zeros((L,), jnp.float32)
    lane = lax.iota(jnp.int32, L)

    def _zrow(r, carry):
        for k in range(128 // L):
            fs_rows[r, pl.ds(k * L, L)] = zf
        return carry
    lax.fori_loop(0, EC, _zrow, 0)
    base = s * RPT
    for blk in range(RPT // EC):
        pltpu.sync_copy(fs_rows, acc_s.at[pl.ds(base + blk * EC, EC)])

    def _zden(i, carry):
        den0_v[pl.ds(i * L, L)] = zf
        den1_v[pl.ds(i * L, L)] = zf
        return carry
    lax.fori_loop(0, NR // L, _zden, 0)

    pltpu.sync_copy(attn_hbm.at[c], attn_v)
    a_vecs = [attn_v[pl.ds(k * L, L)] for k in range(8)]

    coff = jnp.full((L,), c * N, jnp.int32)
    ebase = s * ET1

    plsc.subcore_barrier()

    def _do_chunk():
        cp1 = pltpu.async_copy(fs_hbm.at[src_ch], fs_rows, sem1)
        cp2 = pltpu.async_copy(fd_hbm.at[dstoff_ch], fd_rows, sem2)
        cp1.wait()
        cp2.wait()

        def _group(g, carry):
            dvec = dstrow_ch[pl.ds(g * L, L)]
            for e16 in range(L):
                e = g * L + e16
                fsv = [fs_rows[e, pl.ds(k * L, L)] for k in range(8)]
                fdv = [fd_rows[e, pl.ds(k * L, L)] for k in range(8)]
                av = []
                for k in range(8):
                    ev = fsv[k] + fdv[k]
                    ev = jnp.where(ev > 0, ev, NEG_SLOPE * ev)
                    av.append(ev * a_vecs[k])
                s0 = jnp.sum((av[0] + av[1]) + (av[2] + av[3]))
                s1 = jnp.sum((av[4] + av[5]) + (av[6] + av[7]))
                ex0 = jnp.exp(jnp.full((L,), s0, jnp.float32))
                ex1 = jnp.exp(jnp.full((L,), s1, jnp.float32))
                for k in range(4):
                    fs_rows[e, pl.ds(k * L, L)] = fsv[k] * ex0
                for k in range(4, 8):
                    fs_rows[e, pl.ds(k * L, L)] = fsv[k] * ex1
                plsc.addupdate_scatter(den0_v, [dvec], ex0, mask=lane == e16)
                plsc.addupdate_scatter(den1_v, [dvec], ex1, mask=lane == e16)
            return carry
        lax.fori_loop(0, EC // L, _group, 0)
        pltpu.sync_copy(fs_rows, acc_s.at[dstrow_ch], add=True)

    def _offset_indices():
        for g in range(EC // L):
            src_ch[pl.ds(g * L, L)] = src_ch[pl.ds(g * L, L)] + coff
            dstoff_ch[pl.ds(g * L, L)] = dstrow_ch[pl.ds(g * L, L)] + coff

    def _chunk(j, carry):
        eb = ebase + j * EC
        pltpu.sync_copy(src_hbm.at[pl.ds(eb, EC)], src_ch)
        pltpu.sync_copy(dst_hbm.at[pl.ds(eb, EC)], dstrow_ch)
        _offset_indices()
        _do_chunk()
        return carry
    lax.fori_loop(0, NFULL, _chunk, 0)

    pltpu.sync_copy(src_hbm.at[pl.ds(ebase + NFULL * EC, TAILE)],
                    src_ch.at[pl.ds(0, TAILE)])
    pltpu.sync_copy(dst_hbm.at[pl.ds(ebase + NFULL * EC, TAILE)],
                    dstrow_ch.at[pl.ds(0, TAILE)])
    for t in range(TAILE // L, EC // L):
        src_ch[pl.ds(t * L, L)] = jnp.full((L,), 0, jnp.int32)
        dstrow_ch[pl.ds(t * L, L)] = jnp.full((L,), DUMP, jnp.int32)
    _offset_indices()
    _do_chunk()

    pltpu.sync_copy(den0_v, outden_hbm.at[c, s, 0])
    pltpu.sync_copy(den1_v, outden_hbm.at[c, s, 1])

    plsc.subcore_barrier()
    pltpu.sync_copy(acc_s.at[pl.ds(base, RPT)], out_hbm.at[c, pl.ds(base, RPT)])


def _k2(fs_t, fd_t, src, dst, attn_t):
    f = functools.partial(
        pl.kernel,
        out_type=(
            jax.ShapeDtypeStruct((NC, NR, 128), jnp.float32),
            jax.ShapeDtypeStruct((NC, NS, 2, NR), jnp.float32),
        ),
        mesh=_mesh(),
        compiler_params=pltpu.CompilerParams(needs_layout_passes=False),
        scratch_types=[
            pltpu.VMEM((EC,), jnp.int32),
            pltpu.VMEM((EC,), jnp.int32),
            pltpu.VMEM((EC,), jnp.int32),
            pltpu.VMEM((EC, 128), jnp.float32),
            pltpu.VMEM((EC, 128), jnp.float32),
            pltpu.VMEM((128,), jnp.float32),
            pltpu.VMEM((NR,), jnp.float32),
            pltpu.VMEM((NR,), jnp.float32),
            pltpu.VMEM_SHARED((NR, 128), jnp.float32),
            pltpu.SemaphoreType.DMA,
            pltpu.SemaphoreType.DMA,
        ],
    )(_k2_body)
    return f(fs_t, fd_t, src, dst, attn_t)


def _k3_body(acc_ref, den_ref, b_ref, ws_ref, wd_ref, ft_ref):
    a = acc_ref[...]
    d = den_ref[...]
    parts = []
    for c in range(NC):
        for k in range(2):
            num = a[c, :, k * 64:(k + 1) * 64]
            den = jnp.sum(d[:, (c * 2 + k) * NS:(c * 2 + k + 1) * NS],
                          axis=1, keepdims=True)
            parts.append(num / jnp.maximum(den, 1e-16))
    h = jnp.concatenate(parts, axis=1) + b_ref[...]
    h = jnp.where(h > 0, h, jnp.exp(h) - 1.0)
    ft_ref[...] = jnp.concatenate(
        [jnp.dot(h, ws_ref[...], preferred_element_type=jnp.float32),
         jnp.dot(h, wd_ref[...], preferred_element_type=jnp.float32)], axis=1)


def _k3(acc1, den1, bias1, W_src2, W_dst2):
    nb = 10
    rb = N // nb
    return pl.pallas_call(
        _k3_body,
        grid=(nb,),
        in_specs=[
            pl.BlockSpec((NC, rb, 128), lambda j: (0, j, 0)),
            pl.BlockSpec((rb, NC * 2 * NS), lambda j: (j, 0)),
            pl.BlockSpec((1, HEADS * HID), lambda j: (0, 0)),
            pl.BlockSpec((HEADS * HID, HID), lambda j: (0, 0)),
            pl.BlockSpec((HEADS * HID, HID), lambda j: (0, 0)),
        ],
        out_specs=[
            pl.BlockSpec((rb, 2 * HID), lambda j: (j, 0)),
        ],
        out_shape=[
            jax.ShapeDtypeStruct((N, 2 * HID), jnp.float32),
        ],
    )(acc1, den1, bias1, W_src2, W_dst2)


def _k4_body(ft_hbm, src_hbm, dst_hbm, attn_hbm, out_hbm,
             src_ch, dst_ch, fs_rows, fd_rows, attn_v,
             acc_s, sem1, sem2):
    c = lax.axis_index("c")
    s = lax.axis_index("s")
    zf = jnp.zeros((L,), jnp.float32)
    lane = lax.iota(jnp.int32, L)

    def _zrow(r, carry):
        for k in range(128 // L):
            fs_rows[r, pl.ds(k * L, L)] = zf
        return carry
    lax.fori_loop(0, EC, _zrow, 0)
    base = s * RPT
    for blk in range(RPT // EC):
        pltpu.sync_copy(fs_rows, acc_s.at[pl.ds(base + blk * EC, EC)])

    pltpu.sync_copy(attn_hbm.at[0], attn_v)
    a_vecs = [attn_v[pl.ds(k * L, L)] for k in range(4)]

    ebase = (c * NS + s) * ET2

    plsc.subcore_barrier()

    def _do_chunk():
        cp1 = pltpu.async_copy(ft_hbm.at[src_ch], fs_rows, sem1)
        cp2 = pltpu.async_copy(ft_hbm.at[dst_ch], fd_rows, sem2)
        cp1.wait()
        cp2.wait()

        def _edge(e, carry):
            fsv = [fs_rows[e, pl.ds(k * L, L)] for k in range(4)]
            fdv = [fd_rows[e, pl.ds((4 + k) * L, L)] for k in range(4)]
            av = []
            for k in range(4):
                ev = fsv[k] + fdv[k]
                ev = jnp.where(ev > 0, ev, NEG_SLOPE * ev)
                av.append(ev * a_vecs[k])
            s0 = jnp.sum((av[0] + av[1]) + (av[2] + av[3]))
            ex0 = jnp.exp(jnp.full((L,), s0, jnp.float32))
            for k in range(4):
                fs_rows[e, pl.ds(k * L, L)] = fsv[k] * ex0
            fs_rows[e, pl.ds(64, L)] = jnp.where(lane == 0, ex0, zf)
            for k in range(5, 8):
                fs_rows[e, pl.ds(k * L, L)] = zf
            return carry
        lax.fori_loop(0, EC, _edge, 0)
        pltpu.sync_copy(fs_rows, acc_s.at[dst_ch], add=True)

    def _chunk(j, carry):
        eb = ebase + j * EC
        pltpu.sync_copy(src_hbm.at[pl.ds(eb, EC)], src_ch)
        pltpu.sync_copy(dst_hbm.at[pl.ds(eb, EC)], dst_ch)
        _do_chunk()
        return carry
    lax.fori_loop(0, NF2, _chunk, 0)

    pltpu.sync_copy(src_hbm.at[pl.ds(ebase + NF2 * EC, TAIL2)],
                    src_ch.at[pl.ds(0, TAIL2)])
    pltpu.sync_copy(dst_hbm.at[pl.ds(ebase + NF2 * EC, TAIL2)],
                    dst_ch.at[pl.ds(0, TAIL2)])
    tail_keep = lane < TAIL2
    src_ch[pl.ds(0, L)] = jnp.where(tail_keep, src_ch[pl.ds(0, L)], 0)
    dst_ch[pl.ds(0, L)] = jnp.where(tail_keep, dst_ch[pl.ds(0, L)], DUMP)
    for t in range(1, EC // L):
        src_ch[pl.ds(t * L, L)] = jnp.full((L,), 0, jnp.int32)
        dst_ch[pl.ds(t * L, L)] = jnp.full((L,), DUMP, jnp.int32)
    _do_chunk()

    plsc.subcore_barrier()
    pltpu.sync_copy(acc_s.at[pl.ds(base, RPT)], out_hbm.at[c, pl.ds(base, RPT)])


def _k4(ft2, src, dst, attn2):
    f = functools.partial(
        pl.kernel,
        out_type=jax.ShapeDtypeStruct((NC, NR, 128), jnp.float32),
        mesh=_mesh(),
        compiler_params=pltpu.CompilerParams(needs_layout_passes=False),
        scratch_types=[
            pltpu.VMEM((EC,), jnp.int32),
            pltpu.VMEM((EC,), jnp.int32),
            pltpu.VMEM((EC, 128), jnp.float32),
            pltpu.VMEM((EC, 128), jnp.float32),
            pltpu.VMEM((HID,), jnp.float32),
            pltpu.VMEM_SHARED((NR, 128), jnp.float32),
            pltpu.SemaphoreType.DMA,
            pltpu.SemaphoreType.DMA,
        ],
    )(_k4_body)
    return f(ft2, src, dst, attn2)


def _k5_body(acc_ref, b2_ref,
             wih0, whh0, bih0, bhh0, wih1, whh1, bih1, bhh1,
             wih2, whh2, bih2, bhh2, out_ref):
    a = acc_ref[...]
    num = a[0, :N, :HID] + a[1, :N, :HID]
    den = a[0, :N, HID:HID + 1] + a[1, :N, HID:HID + 1]
    h = num / jnp.maximum(den, 1e-16) + b2_ref[...]
    feat = jnp.where(h > 0, h, jnp.exp(h) - 1.0)

    wihs = (wih0[...], wih1[...], wih2[...])
    whhs = (whh0[...], whh1[...], whh2[...])
    bihs = (bih0[...], bih1[...], bih2[...])
    bhhs = (bhh0[...], bhh1[...], bhh2[...])

    hs = [jnp.zeros((1, HID), jnp.float32) for _ in range(3)]
    cs = [jnp.zeros((1, HID), jnp.float32) for _ in range(3)]
    q_star = jnp.zeros((1, 2 * HID), jnp.float32)

    def dotT(u, w):
        return lax.dot_general(u, w, (((1,), (1,)), ((), ())),
                               preferred_element_type=jnp.float32)

    for _ in range(3):
        inp = q_star
        for l in range(3):
            gates = dotT(inp, wihs[l]) + bihs[l] + dotT(hs[l], whhs[l]) + bhhs[l]
            gi = gates[:, 0:HID]
            gf = gates[:, HID:2 * HID]
            gg = gates[:, 2 * HID:3 * HID]
            go = gates[:, 3 * HID:4 * HID]
            cnew = jax.nn.sigmoid(gf) * cs[l] + jax.nn.sigmoid(gi) * jnp.tanh(gg)
            hnew = jax.nn.sigmoid(go) * jnp.tanh(cnew)
            hs[l] = hnew
            cs[l] = cnew
            inp = hnew
        q = inp
        e = dotT(feat, q)
        m = jnp.max(e)
        z = jnp.exp(e - m)
        ssum = jnp.sum(z)
        r = lax.dot_general(z, feat, (((0,), (0,)), ((), ())),
                            preferred_element_type=jnp.float32) / ssum
        q_star = jnp.concatenate([q, r], axis=1)
    out_ref[...] = q_star


def _k5(acc2, bias2, lstm):
    return pl.pallas_call(
        _k5_body,
        out_shape=jax.ShapeDtypeStruct((1, 2 * HID), jnp.float32),
    )(acc2, bias2, *lstm)


def kernel(x, edge_index, W_src1, W_dst1, attn1, bias1, W_src2, W_dst2,
           attn2, bias2, W_ih0, W_hh0, b_ih0, b_hh0, W_ih1, W_hh1, b_ih1,
           b_hh1, W_ih2, W_hh2, b_ih2, b_hh2):
    src = edge_index[0]
    dst = edge_index[1]
    attn1_t = attn1.reshape(NC, 128)
    bias1_r = bias1.reshape(1, HEADS * HID)
    bias2_r = bias2.reshape(1, HID)

    fs_t, fd_t = _k1(x, W_src1, W_dst1)
    acc1, den1 = _k2(fs_t, fd_t, src, dst, attn1_t)
    den1_t = den1.transpose(3, 0, 2, 1).reshape(NR, NC * 2 * NS)
    ft2, = _k3(acc1, den1_t, bias1_r, W_src2, W_dst2)
    acc2 = _k4(ft2, src, dst, attn2)
    lstm = (W_ih0, W_hh0, b_ih0.reshape(1, -1), b_hh0.reshape(1, -1),
            W_ih1, W_hh1, b_ih1.reshape(1, -1), b_hh1.reshape(1, -1),
            W_ih2, W_hh2, b_ih2.reshape(1, -1), b_hh2.reshape(1, -1))
    return _k5(acc2, bias2_r, lstm)


# speedup vs baseline: 1.3463x; 1.0342x over previous
import functools

import jax
import jax.numpy as jnp
from jax import lax
from jax.experimental import pallas as pl
from jax.experimental.pallas import tpu as pltpu
from jax.experimental.pallas import tpu_sc as plsc

N = 10000
E = 160000
IN_FEAT = 128
HID = 64
HEADS = 4
NEG_SLOPE = 0.2

NC = 2
NS = 16
L = 16

NR = 10240
DUMP = N
RPT = NR // NS

ET1 = E // NS
EC1 = 48
NP1 = 104
TAILE = ET1 - NP1 * 2 * EC1
EC = 64
ET2 = E // (NC * NS)
NF2 = ET2 // EC
TAIL2 = ET2 - NF2 * EC

def _mesh():
    return plsc.VectorSubcoreMesh(core_axis_name="c", subcore_axis_name="s",
                                  num_cores=NC, num_subcores=NS)


def _k1_body(x_ref, ws_ref, wd_ref, fs_ref, fd_ref):
    x = x_ref[...]
    fs_ref[...] = jnp.dot(x, ws_ref[...], preferred_element_type=jnp.float32)
    fd_ref[...] = jnp.dot(x, wd_ref[...], preferred_element_type=jnp.float32)


def _k1(x, W_src1, W_dst1):
    nb = 10
    rb = N // nb
    return pl.pallas_call(
        _k1_body,
        grid=(NC, nb),
        in_specs=[
            pl.BlockSpec((rb, IN_FEAT), lambda c, j: (j, 0)),
            pl.BlockSpec((IN_FEAT, 128), lambda c, j: (0, c)),
            pl.BlockSpec((IN_FEAT, 128), lambda c, j: (0, c)),
        ],
        out_specs=[
            pl.BlockSpec((rb, 128), lambda c, j: (c * nb + j, 0)),
            pl.BlockSpec((rb, 128), lambda c, j: (c * nb + j, 0)),
        ],
        out_shape=[
            jax.ShapeDtypeStruct((NC * N, 128), jnp.float32),
            jax.ShapeDtypeStruct((NC * N, 128), jnp.float32),
        ],
    )(x, W_src1, W_dst1)


def _k2_body(fs_hbm, fd_hbm, src_hbm, dst_hbm, attn_hbm,
             out_hbm, outden_hbm,
             srcA, dofA, drwA, srcB, dofB, drwB,
             fsA, fdA, fsB, fdB, attn_v,
             den0_v, den1_v, acc_s,
             smA1, smA2, smB1, smB2, sgA1, sgA2, sgB1, sgB2):
    c = lax.axis_index("c")
    s = lax.axis_index("s")
    zf = jnp.zeros((L,), jnp.float32)
    lane = lax.iota(jnp.int32, L)

    def _zrow(r, carry):
        for k in range(128 // L):
            fsA[r, pl.ds(k * L, L)] = zf
        return carry
    lax.fori_loop(0, EC1, _zrow, 0)
    base = s * RPT
    for blk in range(RPT // EC1):
        pltpu.sync_copy(fsA, acc_s.at[pl.ds(base + blk * EC1, EC1)])
    remz = RPT - (RPT // EC1) * EC1
    pltpu.sync_copy(fsA.at[pl.ds(0, remz)],
                    acc_s.at[pl.ds(base + (RPT // EC1) * EC1, remz)])

    def _zden(i, carry):
        den0_v[pl.ds(i * L, L)] = zf
        den1_v[pl.ds(i * L, L)] = zf
        return carry
    lax.fori_loop(0, NR // L, _zden, 0)

    pltpu.sync_copy(attn_hbm.at[c], attn_v)
    a_vecs = [attn_v[pl.ds(k * L, L)] for k in range(8)]

    coff = jnp.full((L,), c * N, jnp.int32)
    ebase = s * ET1

    plsc.subcore_barrier()

    def _offsets(src_ch, dof_ch, drw_ch):
        for g in range(EC1 // L):
            src_ch[pl.ds(g * L, L)] = src_ch[pl.ds(g * L, L)] + coff
            dof_ch[pl.ds(g * L, L)] = drw_ch[pl.ds(g * L, L)] + coff

    def _compute(fs_rows, fd_rows, drw_ch):
        def _group(g, carry):
            dvec = drw_ch[pl.ds(g * L, L)]
            for e16 in range(L):
                e = g * L + e16
                fsv = [fs_rows[e, pl.ds(k * L, L)] for k in range(8)]
                fdv = [fd_rows[e, pl.ds(k * L, L)] for k in range(8)]
                av = []
                for k in range(8):
                    ev = fsv[k] + fdv[k]
                    ev = jnp.where(ev > 0, ev, NEG_SLOPE * ev)
                    av.append(ev * a_vecs[k])
                s0 = jnp.sum((av[0] + av[1]) + (av[2] + av[3]))
                s1 = jnp.sum((av[4] + av[5]) + (av[6] + av[7]))
                ex0 = jnp.exp(jnp.full((L,), s0, jnp.float32))
                ex1 = jnp.exp(jnp.full((L,), s1, jnp.float32))
                for k in range(4):
                    fs_rows[e, pl.ds(k * L, L)] = fsv[k] * ex0
                for k in range(4, 8):
                    fs_rows[e, pl.ds(k * L, L)] = fsv[k] * ex1
                plsc.addupdate_scatter(den0_v, [dvec], ex0, mask=lane == e16)
                plsc.addupdate_scatter(den1_v, [dvec], ex1, mask=lane == e16)
            return carry
        lax.fori_loop(0, EC1 // L, _group, 0)
        pltpu.sync_copy(fs_rows, acc_s.at[drw_ch], add=True)

    def _pair(p, carry):
        ebA = ebase + (2 * p) * EC1
        ebB = ebA + EC1
        iA1 = pltpu.async_copy(src_hbm.at[pl.ds(ebA, EC1)], srcA, smA1)
        iA2 = pltpu.async_copy(dst_hbm.at[pl.ds(ebA, EC1)], drwA, smA2)
        iB1 = pltpu.async_copy(src_hbm.at[pl.ds(ebB, EC1)], srcB, smB1)
        iB2 = pltpu.async_copy(dst_hbm.at[pl.ds(ebB, EC1)], drwB, smB2)
        iA1.wait()
        iA2.wait()
        _offsets(srcA, dofA, drwA)
        gA1 = pltpu.async_copy(fs_hbm.at[srcA], fsA, sgA1)
        gA2 = pltpu.async_copy(fd_hbm.at[dofA], fdA, sgA2)
        iB1.wait()
        iB2.wait()
        _offsets(srcB, dofB, drwB)
        gB1 = pltpu.async_copy(fs_hbm.at[srcB], fsB, sgB1)
        gB2 = pltpu.async_copy(fd_hbm.at[dofB], fdB, sgB2)
        gA1.wait()
        gA2.wait()
        _compute(fsA, fdA, drwA)
        gB1.wait()
        gB2.wait()
        _compute(fsB, fdB, drwB)
        return carry
    lax.fori_loop(0, NP1, _pair, 0)

    pltpu.sync_copy(src_hbm.at[pl.ds(ebase + NP1 * 2 * EC1, TAILE)],
                    srcA.at[pl.ds(0, TAILE)])
    pltpu.sync_copy(dst_hbm.at[pl.ds(ebase + NP1 * 2 * EC1, TAILE)],
                    drwA.at[pl.ds(0, TAILE)])
    for t in range(TAILE // L, EC1 // L):
        srcA[pl.ds(t * L, L)] = jnp.full((L,), 0, jnp.int32)
        drwA[pl.ds(t * L, L)] = jnp.full((L,), DUMP, jnp.int32)
    _offsets(srcA, dofA, drwA)
    tA1 = pltpu.async_copy(fs_hbm.at[srcA], fsA, sgA1)
    tA2 = pltpu.async_copy(fd_hbm.at[dofA], fdA, sgA2)
    tA1.wait()
    tA2.wait()
    _compute(fsA, fdA, drwA)

    pltpu.sync_copy(den0_v, outden_hbm.at[c, s, 0])
    pltpu.sync_copy(den1_v, outden_hbm.at[c, s, 1])

    plsc.subcore_barrier()
    pltpu.sync_copy(acc_s.at[pl.ds(base, RPT)], out_hbm.at[c, pl.ds(base, RPT)])


def _k2(fs_t, fd_t, src, dst, attn_t):
    f = functools.partial(
        pl.kernel,
        out_type=(
            jax.ShapeDtypeStruct((NC, NR, 128), jnp.float32),
            jax.ShapeDtypeStruct((NC, NS, 2, NR), jnp.float32),
        ),
        mesh=_mesh(),
        compiler_params=pltpu.CompilerParams(needs_layout_passes=False),
        scratch_types=[
            pltpu.VMEM((EC1,), jnp.int32),
            pltpu.VMEM((EC1,), jnp.int32),
            pltpu.VMEM((EC1,), jnp.int32),
            pltpu.VMEM((EC1,), jnp.int32),
            pltpu.VMEM((EC1,), jnp.int32),
            pltpu.VMEM((EC1,), jnp.int32),
            pltpu.VMEM((EC1, 128), jnp.float32),
            pltpu.VMEM((EC1, 128), jnp.float32),
            pltpu.VMEM((EC1, 128), jnp.float32),
            pltpu.VMEM((EC1, 128), jnp.float32),
            pltpu.VMEM((128,), jnp.float32),
            pltpu.VMEM((NR,), jnp.float32),
            pltpu.VMEM((NR,), jnp.float32),
            pltpu.VMEM_SHARED((NR, 128), jnp.float32),
            pltpu.SemaphoreType.DMA,
            pltpu.SemaphoreType.DMA,
            pltpu.SemaphoreType.DMA,
            pltpu.SemaphoreType.DMA,
            pltpu.SemaphoreType.DMA,
            pltpu.SemaphoreType.DMA,
            pltpu.SemaphoreType.DMA,
            pltpu.SemaphoreType.DMA,
        ],
    )(_k2_body)
    return f(fs_t, fd_t, src, dst, attn_t)


def _k3_body(acc_ref, den_ref, b_ref, ws_ref, wd_ref, ft_ref):
    a = acc_ref[...]
    d = den_ref[...]
    parts = []
    for c in range(NC):
        for k in range(2):
            num = a[c, :, k * 64:(k + 1) * 64]
            den = jnp.sum(d[:, (c * 2 + k) * NS:(c * 2 + k + 1) * NS],
                          axis=1, keepdims=True)
            parts.append(num / jnp.maximum(den, 1e-16))
    h = jnp.concatenate(parts, axis=1) + b_ref[...]
    h = jnp.where(h > 0, h, jnp.exp(h) - 1.0)
    ft_ref[...] = jnp.concatenate(
        [jnp.dot(h, ws_ref[...], preferred_element_type=jnp.float32),
         jnp.dot(h, wd_ref[...], preferred_element_type=jnp.float32)], axis=1)


def _k3(acc1, den1, bias1, W_src2, W_dst2):
    nb = 10
    rb = N // nb
    return pl.pallas_call(
        _k3_body,
        grid=(nb,),
        in_specs=[
            pl.BlockSpec((NC, rb, 128), lambda j: (0, j, 0)),
            pl.BlockSpec((rb, NC * 2 * NS), lambda j: (j, 0)),
            pl.BlockSpec((1, HEADS * HID), lambda j: (0, 0)),
            pl.BlockSpec((HEADS * HID, HID), lambda j: (0, 0)),
            pl.BlockSpec((HEADS * HID, HID), lambda j: (0, 0)),
        ],
        out_specs=[
            pl.BlockSpec((rb, 2 * HID), lambda j: (j, 0)),
        ],
        out_shape=[
            jax.ShapeDtypeStruct((N, 2 * HID), jnp.float32),
        ],
    )(acc1, den1, bias1, W_src2, W_dst2)


def _k4_body(ft_hbm, src_hbm, dst_hbm, attn_hbm, out_hbm,
             src_ch, dst_ch, fs_rows, fd_rows, attn_v,
             acc_s, sem1, sem2):
    c = lax.axis_index("c")
    s = lax.axis_index("s")
    zf = jnp.zeros((L,), jnp.float32)
    lane = lax.iota(jnp.int32, L)

    def _zrow(r, carry):
        for k in range(128 // L):
            fs_rows[r, pl.ds(k * L, L)] = zf
        return carry
    lax.fori_loop(0, EC, _zrow, 0)
    base = s * RPT
    for blk in range(RPT // EC):
        pltpu.sync_copy(fs_rows, acc_s.at[pl.ds(base + blk * EC, EC)])

    pltpu.sync_copy(attn_hbm.at[0], attn_v)
    a_vecs = [attn_v[pl.ds(k * L, L)] for k in range(4)]

    ebase = (c * NS + s) * ET2

    plsc.subcore_barrier()

    def _do_chunk():
        cp1 = pltpu.async_copy(ft_hbm.at[src_ch], fs_rows, sem1)
        cp2 = pltpu.async_copy(ft_hbm.at[dst_ch], fd_rows, sem2)
        cp1.wait()
        cp2.wait()

        def _edge(e, carry):
            fsv = [fs_rows[e, pl.ds(k * L, L)] for k in range(4)]
            fdv = [fd_rows[e, pl.ds((4 + k) * L, L)] for k in range(4)]
            av = []
            for k in range(4):
                ev = fsv[k] + fdv[k]
                ev = jnp.where(ev > 0, ev, NEG_SLOPE * ev)
                av.append(ev * a_vecs[k])
            s0 = jnp.sum((av[0] + av[1]) + (av[2] + av[3]))
            ex0 = jnp.exp(jnp.full((L,), s0, jnp.float32))
            for k in range(4):
                fs_rows[e, pl.ds(k * L, L)] = fsv[k] * ex0
            fs_rows[e, pl.ds(64, L)] = jnp.where(lane == 0, ex0, zf)
            for k in range(5, 8):
                fs_rows[e, pl.ds(k * L, L)] = zf
            return carry
        lax.fori_loop(0, EC, _edge, 0)
        pltpu.sync_copy(fs_rows, acc_s.at[dst_ch], add=True)

    def _chunk(j, carry):
        eb = ebase + j * EC
        pltpu.sync_copy(src_hbm.at[pl.ds(eb, EC)], src_ch)
        pltpu.sync_copy(dst_hbm.at[pl.ds(eb, EC)], dst_ch)
        _do_chunk()
        return carry
    lax.fori_loop(0, NF2, _chunk, 0)

    pltpu.sync_copy(src_hbm.at[pl.ds(ebase + NF2 * EC, TAIL2)],
                    src_ch.at[pl.ds(0, TAIL2)])
    pltpu.sync_copy(dst_hbm.at[pl.ds(ebase + NF2 * EC, TAIL2)],
                    dst_ch.at[pl.ds(0, TAIL2)])
    tail_keep = lane < TAIL2
    src_ch[pl.ds(0, L)] = jnp.where(tail_keep, src_ch[pl.ds(0, L)], 0)
    dst_ch[pl.ds(0, L)] = jnp.where(tail_keep, dst_ch[pl.ds(0, L)], DUMP)
    for t in range(1, EC // L):
        src_ch[pl.ds(t * L, L)] = jnp.full((L,), 0, jnp.int32)
        dst_ch[pl.ds(t * L, L)] = jnp.full((L,), DUMP, jnp.int32)
    _do_chunk()

    plsc.subcore_barrier()
    pltpu.sync_copy(acc_s.at[pl.ds(base, RPT)], out_hbm.at[c, pl.ds(base, RPT)])


def _k4(ft2, src, dst, attn2):
    f = functools.partial(
        pl.kernel,
        out_type=jax.ShapeDtypeStruct((NC, NR, 128), jnp.float32),
        mesh=_mesh(),
        compiler_params=pltpu.CompilerParams(needs_layout_passes=False),
        scratch_types=[
            pltpu.VMEM((EC,), jnp.int32),
            pltpu.VMEM((EC,), jnp.int32),
            pltpu.VMEM((EC, 128), jnp.float32),
            pltpu.VMEM((EC, 128), jnp.float32),
            pltpu.VMEM((HID,), jnp.float32),
            pltpu.VMEM_SHARED((NR, 128), jnp.float32),
            pltpu.SemaphoreType.DMA,
            pltpu.SemaphoreType.DMA,
        ],
    )(_k4_body)
    return f(ft2, src, dst, attn2)


def _k5_body(acc_ref, b2_ref,
             wih0, whh0, bih0, bhh0, wih1, whh1, bih1, bhh1,
             wih2, whh2, bih2, bhh2, out_ref):
    a = acc_ref[...]
    num = a[0, :N, :HID] + a[1, :N, :HID]
    den = a[0, :N, HID:HID + 1] + a[1, :N, HID:HID + 1]
    h = num / jnp.maximum(den, 1e-16) + b2_ref[...]
    feat = jnp.where(h > 0, h, jnp.exp(h) - 1.0)

    wihs = (wih0[...], wih1[...], wih2[...])
    whhs = (whh0[...], whh1[...], whh2[...])
    bihs = (bih0[...], bih1[...], bih2[...])
    bhhs = (bhh0[...], bhh1[...], bhh2[...])

    hs = [jnp.zeros((1, HID), jnp.float32) for _ in range(3)]
    cs = [jnp.zeros((1, HID), jnp.float32) for _ in range(3)]
    q_star = jnp.zeros((1, 2 * HID), jnp.float32)

    def dotT(u, w):
        return lax.dot_general(u, w, (((1,), (1,)), ((), ())),
                               preferred_element_type=jnp.float32)

    for _ in range(3):
        inp = q_star
        for l in range(3):
            gates = dotT(inp, wihs[l]) + bihs[l] + dotT(hs[l], whhs[l]) + bhhs[l]
            gi = gates[:, 0:HID]
            gf = gates[:, HID:2 * HID]
            gg = gates[:, 2 * HID:3 * HID]
            go = gates[:, 3 * HID:4 * HID]
            cnew = jax.nn.sigmoid(gf) * cs[l] + jax.nn.sigmoid(gi) * jnp.tanh(gg)
            hnew = jax.nn.sigmoid(go) * jnp.tanh(cnew)
            hs[l] = hnew
            cs[l] = cnew
            inp = hnew
        q = inp
        e = dotT(feat, q)
        m = jnp.max(e)
        z = jnp.exp(e - m)
        ssum = jnp.sum(z)
        r = lax.dot_general(z, feat, (((0,), (0,)), ((), ())),
                            preferred_element_type=jnp.float32) / ssum
        q_star = jnp.concatenate([q, r], axis=1)
    out_ref[...] = q_star


def _k5(acc2, bias2, lstm):
    return pl.pallas_call(
        _k5_body,
        out_shape=jax.ShapeDtypeStruct((1, 2 * HID), jnp.float32),
    )(acc2, bias2, *lstm)


def kernel(x, edge_index, W_src1, W_dst1, attn1, bias1, W_src2, W_dst2,
           attn2, bias2, W_ih0, W_hh0, b_ih0, b_hh0, W_ih1, W_hh1, b_ih1,
           b_hh1, W_ih2, W_hh2, b_ih2, b_hh2):
    src = edge_index[0]
    dst = edge_index[1]
    attn1_t = attn1.reshape(NC, 128)
    bias1_r = bias1.reshape(1, HEADS * HID)
    bias2_r = bias2.reshape(1, HID)

    fs_t, fd_t = _k1(x, W_src1, W_dst1)
    acc1, den1 = _k2(fs_t, fd_t, src, dst, attn1_t)
    den1_t = den1.transpose(3, 0, 2, 1).reshape(NR, NC * 2 * NS)
    ft2, = _k3(acc1, den1_t, bias1_r, W_src2, W_dst2)
    acc2 = _k4(ft2, src, dst, attn2)
    lstm = (W_ih0, W_hh0, b_ih0.reshape(1, -1), b_hh0.reshape(1, -1),
            W_ih1, W_hh1, b_ih1.reshape(1, -1), b_hh1.reshape(1, -1),
            W_ih2, W_hh2, b_ih2.reshape(1, -1), b_hh2.reshape(1, -1))
    return _k5(acc2, bias2_r, lstm)


# speedup vs baseline: 1.4190x; 1.0540x over previous
import functools

import jax
import jax.numpy as jnp
from jax import lax
from jax.experimental import pallas as pl
from jax.experimental.pallas import tpu as pltpu
from jax.experimental.pallas import tpu_sc as plsc

N = 10000
E = 160000
IN_FEAT = 128
HID = 64
HEADS = 4
NEG_SLOPE = 0.2

NC = 2
NS = 16
L = 16

NR = 10240
DUMP = N
RPT = NR // NS

ET1 = E // NS
EC1 = 48
NP1 = 104
TAILE = ET1 - NP1 * 2 * EC1
EC = 64
ET2 = E // (NC * NS)
NF2 = ET2 // EC
TAIL2 = ET2 - NF2 * EC

def _mesh():
    return plsc.VectorSubcoreMesh(core_axis_name="c", subcore_axis_name="s",
                                  num_cores=NC, num_subcores=NS)


def _k1_body(x_ref, ws_ref, wd_ref, fs_ref, fd_ref):
    x = x_ref[...]
    fs_ref[...] = jnp.dot(x, ws_ref[...], preferred_element_type=jnp.float32)
    fd_ref[...] = jnp.dot(x, wd_ref[...], preferred_element_type=jnp.float32)


def _k1(x, W_src1, W_dst1):
    nb = 10
    rb = N // nb
    return pl.pallas_call(
        _k1_body,
        grid=(NC, nb),
        in_specs=[
            pl.BlockSpec((rb, IN_FEAT), lambda c, j: (j, 0)),
            pl.BlockSpec((IN_FEAT, 128), lambda c, j: (0, c)),
            pl.BlockSpec((IN_FEAT, 128), lambda c, j: (0, c)),
        ],
        out_specs=[
            pl.BlockSpec((rb, 128), lambda c, j: (c * nb + j, 0)),
            pl.BlockSpec((rb, 128), lambda c, j: (c * nb + j, 0)),
        ],
        out_shape=[
            jax.ShapeDtypeStruct((NC * N, 128), jnp.float32),
            jax.ShapeDtypeStruct((NC * N, 128), jnp.float32),
        ],
    )(x, W_src1, W_dst1)


def _k2_body(fs_hbm, fd_hbm, src_hbm, dst_hbm, attn_hbm,
             out_hbm, outden_hbm,
             srcA, dofA, drwA, srcB, dofB, drwB,
             fsA, fdA, fsB, fdB, attn_v,
             den0_v, den1_v, acc_s,
             smA1, smA2, smB1, smB2, sgA1, sgA2, sgB1, sgB2):
    c = lax.axis_index("c")
    s = lax.axis_index("s")
    zf = jnp.zeros((L,), jnp.float32)
    lane = lax.iota(jnp.int32, L)

    def _zrow(r, carry):
        for k in range(128 // L):
            fsA[r, pl.ds(k * L, L)] = zf
        return carry
    lax.fori_loop(0, EC1, _zrow, 0)
    base = s * RPT
    for blk in range(RPT // EC1):
        pltpu.sync_copy(fsA, acc_s.at[pl.ds(base + blk * EC1, EC1)])
    remz = RPT - (RPT // EC1) * EC1
    pltpu.sync_copy(fsA.at[pl.ds(0, remz)],
                    acc_s.at[pl.ds(base + (RPT // EC1) * EC1, remz)])

    def _zden(i, carry):
        den0_v[pl.ds(i * L, L)] = zf
        den1_v[pl.ds(i * L, L)] = zf
        return carry
    lax.fori_loop(0, NR // L, _zden, 0)

    pltpu.sync_copy(attn_hbm.at[c], attn_v)
    a_vecs = [attn_v[pl.ds(k * L, L)] for k in range(8)]

    coff = jnp.full((L,), c * N, jnp.int32)
    ebase = s * ET1

    plsc.subcore_barrier()

    def _offsets(src_ch, dof_ch, drw_ch):
        for g in range(EC1 // L):
            src_ch[pl.ds(g * L, L)] = src_ch[pl.ds(g * L, L)] + coff
            dof_ch[pl.ds(g * L, L)] = drw_ch[pl.ds(g * L, L)] + coff

    def _compute(fs_rows, fd_rows, drw_ch):
        def _group(g, carry):
            dvec = drw_ch[pl.ds(g * L, L)]
            for e16 in range(L):
                e = g * L + e16
                fsv = [fs_rows[e, pl.ds(k * L, L)] for k in range(8)]
                fdv = [fd_rows[e, pl.ds(k * L, L)] for k in range(8)]
                av = []
                for k in range(8):
                    ev = fsv[k] + fdv[k]
                    ev = jnp.where(ev > 0, ev, NEG_SLOPE * ev)
                    av.append(ev * a_vecs[k])
                s0 = jnp.sum((av[0] + av[1]) + (av[2] + av[3]))
                s1 = jnp.sum((av[4] + av[5]) + (av[6] + av[7]))
                ex0 = jnp.exp(jnp.full((L,), s0, jnp.float32))
                ex1 = jnp.exp(jnp.full((L,), s1, jnp.float32))
                for k in range(4):
                    fs_rows[e, pl.ds(k * L, L)] = fsv[k] * ex0
                for k in range(4, 8):
                    fs_rows[e, pl.ds(k * L, L)] = fsv[k] * ex1
                plsc.addupdate_scatter(den0_v, [dvec], ex0, mask=lane == e16)
                plsc.addupdate_scatter(den1_v, [dvec], ex1, mask=lane == e16)
            return carry
        lax.fori_loop(0, EC1 // L, _group, 0)
        pltpu.sync_copy(fs_rows, acc_s.at[drw_ch], add=True)

    def _pair(p, carry):
        ebA = ebase + (2 * p) * EC1
        ebB = ebA + EC1
        iA1 = pltpu.async_copy(src_hbm.at[pl.ds(ebA, EC1)], srcA, smA1)
        iA2 = pltpu.async_copy(dst_hbm.at[pl.ds(ebA, EC1)], drwA, smA2)
        iB1 = pltpu.async_copy(src_hbm.at[pl.ds(ebB, EC1)], srcB, smB1)
        iB2 = pltpu.async_copy(dst_hbm.at[pl.ds(ebB, EC1)], drwB, smB2)
        iA1.wait()
        iA2.wait()
        _offsets(srcA, dofA, drwA)
        gA1 = pltpu.async_copy(fs_hbm.at[srcA], fsA, sgA1)
        gA2 = pltpu.async_copy(fd_hbm.at[dofA], fdA, sgA2)
        iB1.wait()
        iB2.wait()
        _offsets(srcB, dofB, drwB)
        gB1 = pltpu.async_copy(fs_hbm.at[srcB], fsB, sgB1)
        gB2 = pltpu.async_copy(fd_hbm.at[dofB], fdB, sgB2)
        gA1.wait()
        gA2.wait()
        _compute(fsA, fdA, drwA)
        gB1.wait()
        gB2.wait()
        _compute(fsB, fdB, drwB)
        return carry
    lax.fori_loop(0, NP1, _pair, 0)

    pltpu.sync_copy(src_hbm.at[pl.ds(ebase + NP1 * 2 * EC1, TAILE)],
                    srcA.at[pl.ds(0, TAILE)])
    pltpu.sync_copy(dst_hbm.at[pl.ds(ebase + NP1 * 2 * EC1, TAILE)],
                    drwA.at[pl.ds(0, TAILE)])
    for t in range(TAILE // L, EC1 // L):
        srcA[pl.ds(t * L, L)] = jnp.full((L,), 0, jnp.int32)
        drwA[pl.ds(t * L, L)] = jnp.full((L,), DUMP, jnp.int32)
    _offsets(srcA, dofA, drwA)
    tA1 = pltpu.async_copy(fs_hbm.at[srcA], fsA, sgA1)
    tA2 = pltpu.async_copy(fd_hbm.at[dofA], fdA, sgA2)
    tA1.wait()
    tA2.wait()
    _compute(fsA, fdA, drwA)

    pltpu.sync_copy(den0_v, outden_hbm.at[c, s, 0])
    pltpu.sync_copy(den1_v, outden_hbm.at[c, s, 1])

    plsc.subcore_barrier()
    pltpu.sync_copy(acc_s.at[pl.ds(base, RPT)], out_hbm.at[c, pl.ds(base, RPT)])


def _k2(fs_t, fd_t, src, dst, attn_t):
    f = functools.partial(
        pl.kernel,
        out_type=(
            jax.ShapeDtypeStruct((NC, NR, 128), jnp.float32),
            jax.ShapeDtypeStruct((NC, NS, 2, NR), jnp.float32),
        ),
        mesh=_mesh(),
        compiler_params=pltpu.CompilerParams(needs_layout_passes=False),
        scratch_types=[
            pltpu.VMEM((EC1,), jnp.int32),
            pltpu.VMEM((EC1,), jnp.int32),
            pltpu.VMEM((EC1,), jnp.int32),
            pltpu.VMEM((EC1,), jnp.int32),
            pltpu.VMEM((EC1,), jnp.int32),
            pltpu.VMEM((EC1,), jnp.int32),
            pltpu.VMEM((EC1, 128), jnp.float32),
            pltpu.VMEM((EC1, 128), jnp.float32),
            pltpu.VMEM((EC1, 128), jnp.float32),
            pltpu.VMEM((EC1, 128), jnp.float32),
            pltpu.VMEM((128,), jnp.float32),
            pltpu.VMEM((NR,), jnp.float32),
            pltpu.VMEM((NR,), jnp.float32),
            pltpu.VMEM_SHARED((NR, 128), jnp.float32),
            pltpu.SemaphoreType.DMA,
            pltpu.SemaphoreType.DMA,
            pltpu.SemaphoreType.DMA,
            pltpu.SemaphoreType.DMA,
            pltpu.SemaphoreType.DMA,
            pltpu.SemaphoreType.DMA,
            pltpu.SemaphoreType.DMA,
            pltpu.SemaphoreType.DMA,
        ],
    )(_k2_body)
    return f(fs_t, fd_t, src, dst, attn_t)


def _k3_body(acc_ref, den_ref, b_ref, ws_ref, wd_ref, ft_ref):
    a = acc_ref[...]
    d = den_ref[...]
    parts = []
    for c in range(NC):
        for k in range(2):
            num = a[c, :, k * 64:(k + 1) * 64]
            den = jnp.sum(d[:, (c * 2 + k) * NS:(c * 2 + k + 1) * NS],
                          axis=1, keepdims=True)
            parts.append(num / jnp.maximum(den, 1e-16))
    h = jnp.concatenate(parts, axis=1) + b_ref[...]
    h = jnp.where(h > 0, h, jnp.exp(h) - 1.0)
    ft_ref[...] = jnp.concatenate(
        [jnp.dot(h, ws_ref[...], preferred_element_type=jnp.float32),
         jnp.dot(h, wd_ref[...], preferred_element_type=jnp.float32)], axis=1)


def _k3(acc1, den1, bias1, W_src2, W_dst2):
    nb = 10
    rb = N // nb
    return pl.pallas_call(
        _k3_body,
        grid=(nb,),
        in_specs=[
            pl.BlockSpec((NC, rb, 128), lambda j: (0, j, 0)),
            pl.BlockSpec((rb, NC * 2 * NS), lambda j: (j, 0)),
            pl.BlockSpec((1, HEADS * HID), lambda j: (0, 0)),
            pl.BlockSpec((HEADS * HID, HID), lambda j: (0, 0)),
            pl.BlockSpec((HEADS * HID, HID), lambda j: (0, 0)),
        ],
        out_specs=[
            pl.BlockSpec((rb, 2 * HID), lambda j: (j, 0)),
        ],
        out_shape=[
            jax.ShapeDtypeStruct((N, 2 * HID), jnp.float32),
        ],
    )(acc1, den1, bias1, W_src2, W_dst2)


def _k4_body(ft_hbm, src_hbm, dst_hbm, attn_hbm, out_hbm,
             srcA, dstA, srcB, dstB, fsA, fdA, fsB, fdB, attn_v,
             acc_s, smA1, smA2, smB1, smB2, sgA1, sgA2, sgB1, sgB2):
    c = lax.axis_index("c")
    s = lax.axis_index("s")
    zf = jnp.zeros((L,), jnp.float32)
    lane = lax.iota(jnp.int32, L)

    def _zrow(r, carry):
        for k in range(128 // L):
            fsA[r, pl.ds(k * L, L)] = zf
        return carry
    lax.fori_loop(0, EC, _zrow, 0)
    base = s * RPT
    for blk in range(RPT // EC):
        pltpu.sync_copy(fsA, acc_s.at[pl.ds(base + blk * EC, EC)])

    pltpu.sync_copy(attn_hbm.at[0], attn_v)
    a_vecs = [attn_v[pl.ds(k * L, L)] for k in range(4)]

    ebase = (c * NS + s) * ET2

    plsc.subcore_barrier()

    def _compute(fs_rows, fd_rows, dst_ch):
        def _edge(e, carry):
            fsv = [fs_rows[e, pl.ds(k * L, L)] for k in range(4)]
            fdv = [fd_rows[e, pl.ds((4 + k) * L, L)] for k in range(4)]
            av = []
            for k in range(4):
                ev = fsv[k] + fdv[k]
                ev = jnp.where(ev > 0, ev, NEG_SLOPE * ev)
                av.append(ev * a_vecs[k])
            s0 = jnp.sum((av[0] + av[1]) + (av[2] + av[3]))
            ex0 = jnp.exp(jnp.full((L,), s0, jnp.float32))
            for k in range(4):
                fs_rows[e, pl.ds(k * L, L)] = fsv[k] * ex0
            fs_rows[e, pl.ds(64, L)] = jnp.where(lane == 0, ex0, zf)
            for k in range(5, 8):
                fs_rows[e, pl.ds(k * L, L)] = zf
            return carry
        lax.fori_loop(0, EC, _edge, 0)
        pltpu.sync_copy(fs_rows, acc_s.at[dst_ch], add=True)

    def _pair(p, carry):
        ebA = ebase + (2 * p) * EC
        ebB = ebA + EC
        iA1 = pltpu.async_copy(src_hbm.at[pl.ds(ebA, EC)], srcA, smA1)
        iA2 = pltpu.async_copy(dst_hbm.at[pl.ds(ebA, EC)], dstA, smA2)
        iB1 = pltpu.async_copy(src_hbm.at[pl.ds(ebB, EC)], srcB, smB1)
        iB2 = pltpu.async_copy(dst_hbm.at[pl.ds(ebB, EC)], dstB, smB2)
        iA1.wait()
        iA2.wait()
        gA1 = pltpu.async_copy(ft_hbm.at[srcA], fsA, sgA1)
        gA2 = pltpu.async_copy(ft_hbm.at[dstA], fdA, sgA2)
        iB1.wait()
        iB2.wait()
        gB1 = pltpu.async_copy(ft_hbm.at[srcB], fsB, sgB1)
        gB2 = pltpu.async_copy(ft_hbm.at[dstB], fdB, sgB2)
        gA1.wait()
        gA2.wait()
        _compute(fsA, fdA, dstA)
        gB1.wait()
        gB2.wait()
        _compute(fsB, fdB, dstB)
        return carry
    lax.fori_loop(0, NF2 // 2, _pair, 0)

    pltpu.sync_copy(src_hbm.at[pl.ds(ebase + NF2 * EC, TAIL2)],
                    srcA.at[pl.ds(0, TAIL2)])
    pltpu.sync_copy(dst_hbm.at[pl.ds(ebase + NF2 * EC, TAIL2)],
                    dstA.at[pl.ds(0, TAIL2)])
    tail_keep = lane < TAIL2
    srcA[pl.ds(0, L)] = jnp.where(tail_keep, srcA[pl.ds(0, L)], 0)
    dstA[pl.ds(0, L)] = jnp.where(tail_keep, dstA[pl.ds(0, L)], DUMP)
    for t in range(1, EC // L):
        srcA[pl.ds(t * L, L)] = jnp.full((L,), 0, jnp.int32)
        dstA[pl.ds(t * L, L)] = jnp.full((L,), DUMP, jnp.int32)
    tA1 = pltpu.async_copy(ft_hbm.at[srcA], fsA, sgA1)
    tA2 = pltpu.async_copy(ft_hbm.at[dstA], fdA, sgA2)
    tA1.wait()
    tA2.wait()
    _compute(fsA, fdA, dstA)

    plsc.subcore_barrier()
    pltpu.sync_copy(acc_s.at[pl.ds(base, RPT)], out_hbm.at[c, pl.ds(base, RPT)])


def _k4(ft2, src, dst, attn2):
    f = functools.partial(
        pl.kernel,
        out_type=jax.ShapeDtypeStruct((NC, NR, 128), jnp.float32),
        mesh=_mesh(),
        compiler_params=pltpu.CompilerParams(needs_layout_passes=False),
        scratch_types=[
            pltpu.VMEM((EC,), jnp.int32),
            pltpu.VMEM((EC,), jnp.int32),
            pltpu.VMEM((EC,), jnp.int32),
            pltpu.VMEM((EC,), jnp.int32),
            pltpu.VMEM((EC, 128), jnp.float32),
            pltpu.VMEM((EC, 128), jnp.float32),
            pltpu.VMEM((EC, 128), jnp.float32),
            pltpu.VMEM((EC, 128), jnp.float32),
            pltpu.VMEM((HID,), jnp.float32),
            pltpu.VMEM_SHARED((NR, 128), jnp.float32),
            pltpu.SemaphoreType.DMA,
            pltpu.SemaphoreType.DMA,
            pltpu.SemaphoreType.DMA,
            pltpu.SemaphoreType.DMA,
            pltpu.SemaphoreType.DMA,
            pltpu.SemaphoreType.DMA,
            pltpu.SemaphoreType.DMA,
            pltpu.SemaphoreType.DMA,
        ],
    )(_k4_body)
    return f(ft2, src, dst, attn2)


def _k5_body(acc_ref, b2_ref,
             wih0, whh0, bih0, bhh0, wih1, whh1, bih1, bhh1,
             wih2, whh2, bih2, bhh2, out_ref):
    a = acc_ref[...]
    num = a[0, :N, :HID] + a[1, :N, :HID]
    den = a[0, :N, HID:HID + 1] + a[1, :N, HID:HID + 1]
    h = num / jnp.maximum(den, 1e-16) + b2_ref[...]
    feat = jnp.where(h > 0, h, jnp.exp(h) - 1.0)

    wihs = (wih0[...], wih1[...], wih2[...])
    whhs = (whh0[...], whh1[...], whh2[...])
    bihs = (bih0[...], bih1[...], bih2[...])
    bhhs = (bhh0[...], bhh1[...], bhh2[...])

    hs = [jnp.zeros((1, HID), jnp.float32) for _ in range(3)]
    cs = [jnp.zeros((1, HID), jnp.float32) for _ in range(3)]
    q_star = jnp.zeros((1, 2 * HID), jnp.float32)

    def dotT(u, w):
        return lax.dot_general(u, w, (((1,), (1,)), ((), ())),
                               preferred_element_type=jnp.float32)

    for _ in range(3):
        inp = q_star
        for l in range(3):
            gates = dotT(inp, wihs[l]) + bihs[l] + dotT(hs[l], whhs[l]) + bhhs[l]
            gi = gates[:, 0:HID]
            gf = gates[:, HID:2 * HID]
            gg = gates[:, 2 * HID:3 * HID]
            go = gates[:, 3 * HID:4 * HID]
            cnew = jax.nn.sigmoid(gf) * cs[l] + jax.nn.sigmoid(gi) * jnp.tanh(gg)
            hnew = jax.nn.sigmoid(go) * jnp.tanh(cnew)
            hs[l] = hnew
            cs[l] = cnew
            inp = hnew
        q = inp
        e = dotT(feat, q)
        m = jnp.max(e)
        z = jnp.exp(e - m)
        ssum = jnp.sum(z)
        r = lax.dot_general(z, feat, (((0,), (0,)), ((), ())),
                            preferred_element_type=jnp.float32) / ssum
        q_star = jnp.concatenate([q, r], axis=1)
    out_ref[...] = q_star


def _k5(acc2, bias2, lstm):
    return pl.pallas_call(
        _k5_body,
        out_shape=jax.ShapeDtypeStruct((1, 2 * HID), jnp.float32),
    )(acc2, bias2, *lstm)


def kernel(x, edge_index, W_src1, W_dst1, attn1, bias1, W_src2, W_dst2,
           attn2, bias2, W_ih0, W_hh0, b_ih0, b_hh0, W_ih1, W_hh1, b_ih1,
           b_hh1, W_ih2, W_hh2, b_ih2, b_hh2):
    src = edge_index[0]
    dst = edge_index[1]
    attn1_t = attn1.reshape(NC, 128)
    bias1_r = bias1.reshape(1, HEADS * HID)
    bias2_r = bias2.reshape(1, HID)

    fs_t, fd_t = _k1(x, W_src1, W_dst1)
    acc1, den1 = _k2(fs_t, fd_t, src, dst, attn1_t)
    den1_t = den1.transpose(3, 0, 2, 1).reshape(NR, NC * 2 * NS)
    ft2, = _k3(acc1, den1_t, bias1_r, W_src2, W_dst2)
    acc2 = _k4(ft2, src, dst, attn2)
    lstm = (W_ih0, W_hh0, b_ih0.reshape(1, -1), b_hh0.reshape(1, -1),
            W_ih1, W_hh1, b_ih1.reshape(1, -1), b_hh1.reshape(1, -1),
            W_ih2, W_hh2, b_ih2.reshape(1, -1), b_hh2.reshape(1, -1))
    return _k5(acc2, bias2_r, lstm)


# speedup vs baseline: 1.4469x; 1.0197x over previous
import functools

import jax
import jax.numpy as jnp
from jax import lax
from jax.experimental import pallas as pl
from jax.experimental.pallas import tpu as pltpu
from jax.experimental.pallas import tpu_sc as plsc

N = 10000
E = 160000
IN_FEAT = 128
HID = 64
HEADS = 4
NEG_SLOPE = 0.2

NC = 2
NS = 16
L = 16

NR = 10240
DUMP = N
RPT = NR // NS

ET1 = E // NS
EC1 = 48
NP1 = 104
TAILE = ET1 - NP1 * 2 * EC1
EC = 64
ET2 = E // (NC * NS)
NF2 = ET2 // EC
TAIL2 = ET2 - NF2 * EC

def _mesh():
    return plsc.VectorSubcoreMesh(core_axis_name="c", subcore_axis_name="s",
                                  num_cores=NC, num_subcores=NS)


def _k1_body(x_ref, ws_ref, wd_ref, fs_ref, fd_ref):
    x = x_ref[...]
    fs_ref[...] = jnp.dot(x, ws_ref[...], preferred_element_type=jnp.float32)
    fd_ref[...] = jnp.dot(x, wd_ref[...], preferred_element_type=jnp.float32)


def _k1(x, W_src1, W_dst1):
    nb = 10
    rb = N // nb
    return pl.pallas_call(
        _k1_body,
        grid=(NC, nb),
        in_specs=[
            pl.BlockSpec((rb, IN_FEAT), lambda c, j: (j, 0)),
            pl.BlockSpec((IN_FEAT, 128), lambda c, j: (0, c)),
            pl.BlockSpec((IN_FEAT, 128), lambda c, j: (0, c)),
        ],
        out_specs=[
            pl.BlockSpec((rb, 128), lambda c, j: (c * nb + j, 0)),
            pl.BlockSpec((rb, 128), lambda c, j: (c * nb + j, 0)),
        ],
        out_shape=[
            jax.ShapeDtypeStruct((NC * N, 128), jnp.float32),
            jax.ShapeDtypeStruct((NC * N, 128), jnp.float32),
        ],
    )(x, W_src1, W_dst1)


def _k2_body(fs_hbm, fd_hbm, src_hbm, dst_hbm, attn_hbm,
             out_hbm, outden_hbm,
             srcA, dofA, drwA, srcB, dofB, drwB,
             fsA, fdA, fsB, fdB, attn_v,
             den0_v, den1_v, acc_s,
             smA1, smA2, smB1, smB2, sgA1, sgA2, sgB1, sgB2):
    c = lax.axis_index("c")
    s = lax.axis_index("s")
    zf = jnp.zeros((L,), jnp.float32)
    lane = lax.iota(jnp.int32, L)

    def _zrow(r, carry):
        for k in range(128 // L):
            fsA[r, pl.ds(k * L, L)] = zf
        return carry
    lax.fori_loop(0, EC1, _zrow, 0)
    base = s * RPT
    for blk in range(RPT // EC1):
        pltpu.sync_copy(fsA, acc_s.at[pl.ds(base + blk * EC1, EC1)])
    remz = RPT - (RPT // EC1) * EC1
    pltpu.sync_copy(fsA.at[pl.ds(0, remz)],
                    acc_s.at[pl.ds(base + (RPT // EC1) * EC1, remz)])

    def _zden(i, carry):
        den0_v[pl.ds(i * L, L)] = zf
        den1_v[pl.ds(i * L, L)] = zf
        return carry
    lax.fori_loop(0, NR // L, _zden, 0)

    pltpu.sync_copy(attn_hbm.at[c], attn_v)
    a_vecs = [attn_v[pl.ds(k * L, L)] for k in range(8)]
    an_vecs = [a * NEG_SLOPE for a in a_vecs]
    masks = [lane == i for i in range(L)]

    coff = jnp.full((L,), c * N, jnp.int32)
    ebase = s * ET1

    plsc.subcore_barrier()

    def _offsets(src_ch, dof_ch, drw_ch):
        for g in range(EC1 // L):
            src_ch[pl.ds(g * L, L)] = src_ch[pl.ds(g * L, L)] + coff
            dof_ch[pl.ds(g * L, L)] = drw_ch[pl.ds(g * L, L)] + coff

    def _compute(fs_rows, fd_rows, drw_ch):
        def _group(g, carry):
            dvec = drw_ch[pl.ds(g * L, L)]
            for e16 in range(L):
                e = g * L + e16
                fsv = [fs_rows[e, pl.ds(k * L, L)] for k in range(8)]
                fdv = [fd_rows[e, pl.ds(k * L, L)] for k in range(8)]
                av = []
                for k in range(8):
                    ev = fsv[k] + fdv[k]
                    av.append(ev * jnp.where(ev > 0, a_vecs[k], an_vecs[k]))
                s0 = jnp.sum((av[0] + av[1]) + (av[2] + av[3]))
                s1 = jnp.sum((av[4] + av[5]) + (av[6] + av[7]))
                ex0 = jnp.exp(jnp.full((L,), s0, jnp.float32))
                ex1 = jnp.exp(jnp.full((L,), s1, jnp.float32))
                for k in range(4):
                    fs_rows[e, pl.ds(k * L, L)] = fsv[k] * ex0
                for k in range(4, 8):
                    fs_rows[e, pl.ds(k * L, L)] = fsv[k] * ex1
                plsc.addupdate_scatter(den0_v, [dvec], ex0, mask=masks[e16])
                plsc.addupdate_scatter(den1_v, [dvec], ex1, mask=masks[e16])
            return carry
        lax.fori_loop(0, EC1 // L, _group, 0)
        pltpu.sync_copy(fs_rows, acc_s.at[drw_ch], add=True)

    def _pair(p, carry):
        ebA = ebase + (2 * p) * EC1
        ebB = ebA + EC1
        iA1 = pltpu.async_copy(src_hbm.at[pl.ds(ebA, EC1)], srcA, smA1)
        iA2 = pltpu.async_copy(dst_hbm.at[pl.ds(ebA, EC1)], drwA, smA2)
        iB1 = pltpu.async_copy(src_hbm.at[pl.ds(ebB, EC1)], srcB, smB1)
        iB2 = pltpu.async_copy(dst_hbm.at[pl.ds(ebB, EC1)], drwB, smB2)
        iA1.wait()
        iA2.wait()
        _offsets(srcA, dofA, drwA)
        gA1 = pltpu.async_copy(fs_hbm.at[srcA], fsA, sgA1)
        gA2 = pltpu.async_copy(fd_hbm.at[dofA], fdA, sgA2)
        iB1.wait()
        iB2.wait()
        _offsets(srcB, dofB, drwB)
        gB1 = pltpu.async_copy(fs_hbm.at[srcB], fsB, sgB1)
        gB2 = pltpu.async_copy(fd_hbm.at[dofB], fdB, sgB2)
        gA1.wait()
        gA2.wait()
        _compute(fsA, fdA, drwA)
        gB1.wait()
        gB2.wait()
        _compute(fsB, fdB, drwB)
        return carry
    lax.fori_loop(0, NP1, _pair, 0)

    pltpu.sync_copy(src_hbm.at[pl.ds(ebase + NP1 * 2 * EC1, TAILE)],
                    srcA.at[pl.ds(0, TAILE)])
    pltpu.sync_copy(dst_hbm.at[pl.ds(ebase + NP1 * 2 * EC1, TAILE)],
                    drwA.at[pl.ds(0, TAILE)])
    for t in range(TAILE // L, EC1 // L):
        srcA[pl.ds(t * L, L)] = jnp.full((L,), 0, jnp.int32)
        drwA[pl.ds(t * L, L)] = jnp.full((L,), DUMP, jnp.int32)
    _offsets(srcA, dofA, drwA)
    tA1 = pltpu.async_copy(fs_hbm.at[srcA], fsA, sgA1)
    tA2 = pltpu.async_copy(fd_hbm.at[dofA], fdA, sgA2)
    tA1.wait()
    tA2.wait()
    _compute(fsA, fdA, drwA)

    pltpu.sync_copy(den0_v, outden_hbm.at[c, s, 0])
    pltpu.sync_copy(den1_v, outden_hbm.at[c, s, 1])

    plsc.subcore_barrier()
    pltpu.sync_copy(acc_s.at[pl.ds(base, RPT)], out_hbm.at[c, pl.ds(base, RPT)])


def _k2(fs_t, fd_t, src, dst, attn_t):
    f = functools.partial(
        pl.kernel,
        out_type=(
            jax.ShapeDtypeStruct((NC, NR, 128), jnp.float32),
            jax.ShapeDtypeStruct((NC, NS, 2, NR), jnp.float32),
        ),
        mesh=_mesh(),
        compiler_params=pltpu.CompilerParams(needs_layout_passes=False),
        scratch_types=[
            pltpu.VMEM((EC1,), jnp.int32),
            pltpu.VMEM((EC1,), jnp.int32),
            pltpu.VMEM((EC1,), jnp.int32),
            pltpu.VMEM((EC1,), jnp.int32),
            pltpu.VMEM((EC1,), jnp.int32),
            pltpu.VMEM((EC1,), jnp.int32),
            pltpu.VMEM((EC1, 128), jnp.float32),
            pltpu.VMEM((EC1, 128), jnp.float32),
            pltpu.VMEM((EC1, 128), jnp.float32),
            pltpu.VMEM((EC1, 128), jnp.float32),
            pltpu.VMEM((128,), jnp.float32),
            pltpu.VMEM((NR,), jnp.float32),
            pltpu.VMEM((NR,), jnp.float32),
            pltpu.VMEM_SHARED((NR, 128), jnp.float32),
            pltpu.SemaphoreType.DMA,
            pltpu.SemaphoreType.DMA,
            pltpu.SemaphoreType.DMA,
            pltpu.SemaphoreType.DMA,
            pltpu.SemaphoreType.DMA,
            pltpu.SemaphoreType.DMA,
            pltpu.SemaphoreType.DMA,
            pltpu.SemaphoreType.DMA,
        ],
    )(_k2_body)
    return f(fs_t, fd_t, src, dst, attn_t)


def _k3_body(acc_ref, den_ref, b_ref, ws_ref, wd_ref, ft_ref):
    a = acc_ref[...]
    d = den_ref[...]
    parts = []
    for c in range(NC):
        for k in range(2):
            num = a[c, :, k * 64:(k + 1) * 64]
            den = jnp.sum(d[:, (c * 2 + k) * NS:(c * 2 + k + 1) * NS],
                          axis=1, keepdims=True)
            parts.append(num / jnp.maximum(den, 1e-16))
    h = jnp.concatenate(parts, axis=1) + b_ref[...]
    h = jnp.where(h > 0, h, jnp.exp(h) - 1.0)
    ft_ref[...] = jnp.concatenate(
        [jnp.dot(h, ws_ref[...], preferred_element_type=jnp.float32),
         jnp.dot(h, wd_ref[...], preferred_element_type=jnp.float32)], axis=1)


def _k3(acc1, den1, bias1, W_src2, W_dst2):
    nb = 10
    rb = N // nb
    return pl.pallas_call(
        _k3_body,
        grid=(nb,),
        in_specs=[
            pl.BlockSpec((NC, rb, 128), lambda j: (0, j, 0)),
            pl.BlockSpec((rb, NC * 2 * NS), lambda j: (j, 0)),
            pl.BlockSpec((1, HEADS * HID), lambda j: (0, 0)),
            pl.BlockSpec((HEADS * HID, HID), lambda j: (0, 0)),
            pl.BlockSpec((HEADS * HID, HID), lambda j: (0, 0)),
        ],
        out_specs=[
            pl.BlockSpec((rb, 2 * HID), lambda j: (j, 0)),
        ],
        out_shape=[
            jax.ShapeDtypeStruct((N, 2 * HID), jnp.float32),
        ],
    )(acc1, den1, bias1, W_src2, W_dst2)


def _k4_body(ft_hbm, src_hbm, dst_hbm, attn_hbm, out_hbm,
             srcA, dstA, srcB, dstB, fsA, fdA, fsB, fdB, attn_v,
             acc_s, smA1, smA2, smB1, smB2, sgA1, sgA2, sgB1, sgB2):
    c = lax.axis_index("c")
    s = lax.axis_index("s")
    zf = jnp.zeros((L,), jnp.float32)
    lane = lax.iota(jnp.int32, L)

    def _zrow(r, carry):
        for k in range(128 // L):
            fsA[r, pl.ds(k * L, L)] = zf
        return carry
    lax.fori_loop(0, EC, _zrow, 0)
    base = s * RPT
    for blk in range(RPT // EC):
        pltpu.sync_copy(fsA, acc_s.at[pl.ds(base + blk * EC, EC)])

    pltpu.sync_copy(attn_hbm.at[0], attn_v)
    a_vecs = [attn_v[pl.ds(k * L, L)] for k in range(4)]
    an_vecs = [a * NEG_SLOPE for a in a_vecs]

    ebase = (c * NS + s) * ET2

    plsc.subcore_barrier()

    def _compute(fs_rows, fd_rows, dst_ch):
        def _edge(e, carry):
            fsv = [fs_rows[e, pl.ds(k * L, L)] for k in range(4)]
            fdv = [fd_rows[e, pl.ds((4 + k) * L, L)] for k in range(4)]
            av = []
            for k in range(4):
                ev = fsv[k] + fdv[k]
                av.append(ev * jnp.where(ev > 0, a_vecs[k], an_vecs[k]))
            s0 = jnp.sum((av[0] + av[1]) + (av[2] + av[3]))
            ex0 = jnp.exp(jnp.full((L,), s0, jnp.float32))
            for k in range(4):
                fs_rows[e, pl.ds(k * L, L)] = fsv[k] * ex0
            fs_rows[e, pl.ds(64, L)] = jnp.where(lane == 0, ex0, zf)
            for k in range(5, 8):
                fs_rows[e, pl.ds(k * L, L)] = zf
            return carry
        lax.fori_loop(0, EC, _edge, 0)
        pltpu.sync_copy(fs_rows, acc_s.at[dst_ch], add=True)

    def _pair(p, carry):
        ebA = ebase + (2 * p) * EC
        ebB = ebA + EC
        iA1 = pltpu.async_copy(src_hbm.at[pl.ds(ebA, EC)], srcA, smA1)
        iA2 = pltpu.async_copy(dst_hbm.at[pl.ds(ebA, EC)], dstA, smA2)
        iB1 = pltpu.async_copy(src_hbm.at[pl.ds(ebB, EC)], srcB, smB1)
        iB2 = pltpu.async_copy(dst_hbm.at[pl.ds(ebB, EC)], dstB, smB2)
        iA1.wait()
        iA2.wait()
        gA1 = pltpu.async_copy(ft_hbm.at[srcA], fsA, sgA1)
        gA2 = pltpu.async_copy(ft_hbm.at[dstA], fdA, sgA2)
        iB1.wait()
        iB2.wait()
        gB1 = pltpu.async_copy(ft_hbm.at[srcB], fsB, sgB1)
        gB2 = pltpu.async_copy(ft_hbm.at[dstB], fdB, sgB2)
        gA1.wait()
        gA2.wait()
        _compute(fsA, fdA, dstA)
        gB1.wait()
        gB2.wait()
        _compute(fsB, fdB, dstB)
        return carry
    lax.fori_loop(0, NF2 // 2, _pair, 0)

    pltpu.sync_copy(src_hbm.at[pl.ds(ebase + NF2 * EC, TAIL2)],
                    srcA.at[pl.ds(0, TAIL2)])
    pltpu.sync_copy(dst_hbm.at[pl.ds(ebase + NF2 * EC, TAIL2)],
                    dstA.at[pl.ds(0, TAIL2)])
    tail_keep = lane < TAIL2
    srcA[pl.ds(0, L)] = jnp.where(tail_keep, srcA[pl.ds(0, L)], 0)
    dstA[pl.ds(0, L)] = jnp.where(tail_keep, dstA[pl.ds(0, L)], DUMP)
    for t in range(1, EC // L):
        srcA[pl.ds(t * L, L)] = jnp.full((L,), 0, jnp.int32)
        dstA[pl.ds(t * L, L)] = jnp.full((L,), DUMP, jnp.int32)
    tA1 = pltpu.async_copy(ft_hbm.at[srcA], fsA, sgA1)
    tA2 = pltpu.async_copy(ft_hbm.at[dstA], fdA, sgA2)
    tA1.wait()
    tA2.wait()
    _compute(fsA, fdA, dstA)

    plsc.subcore_barrier()
    pltpu.sync_copy(acc_s.at[pl.ds(base, RPT)], out_hbm.at[c, pl.ds(base, RPT)])


def _k4(ft2, src, dst, attn2):
    f = functools.partial(
        pl.kernel,
        out_type=jax.ShapeDtypeStruct((NC, NR, 128), jnp.float32),
        mesh=_mesh(),
        compiler_params=pltpu.CompilerParams(needs_layout_passes=False),
        scratch_types=[
            pltpu.VMEM((EC,), jnp.int32),
            pltpu.VMEM((EC,), jnp.int32),
            pltpu.VMEM((EC,), jnp.int32),
            pltpu.VMEM((EC,), jnp.int32),
            pltpu.VMEM((EC, 128), jnp.float32),
            pltpu.VMEM((EC, 128), jnp.float32),
            pltpu.VMEM((EC, 128), jnp.float32),
            pltpu.VMEM((EC, 128), jnp.float32),
            pltpu.VMEM((HID,), jnp.float32),
            pltpu.VMEM_SHARED((NR, 128), jnp.float32),
            pltpu.SemaphoreType.DMA,
            pltpu.SemaphoreType.DMA,
            pltpu.SemaphoreType.DMA,
            pltpu.SemaphoreType.DMA,
            pltpu.SemaphoreType.DMA,
            pltpu.SemaphoreType.DMA,
            pltpu.SemaphoreType.DMA,
            pltpu.SemaphoreType.DMA,
        ],
    )(_k4_body)
    return f(ft2, src, dst, attn2)


def _k5_body(acc_ref, b2_ref,
             wih0, whh0, bih0, bhh0, wih1, whh1, bih1, bhh1,
             wih2, whh2, bih2, bhh2, out_ref):
    a = acc_ref[...]
    num = a[0, :N, :HID] + a[1, :N, :HID]
    den = a[0, :N, HID:HID + 1] + a[1, :N, HID:HID + 1]
    h = num / jnp.maximum(den, 1e-16) + b2_ref[...]
    feat = jnp.where(h > 0, h, jnp.exp(h) - 1.0)

    wihs = (wih0[...], wih1[...], wih2[...])
    whhs = (whh0[...], whh1[...], whh2[...])
    bihs = (bih0[...], bih1[...], bih2[...])
    bhhs = (bhh0[...], bhh1[...], bhh2[...])

    hs = [jnp.zeros((1, HID), jnp.float32) for _ in range(3)]
    cs = [jnp.zeros((1, HID), jnp.float32) for _ in range(3)]
    q_star = jnp.zeros((1, 2 * HID), jnp.float32)

    def dotT(u, w):
        return lax.dot_general(u, w, (((1,), (1,)), ((), ())),
                               preferred_element_type=jnp.float32)

    for _ in range(3):
        inp = q_star
        for l in range(3):
            gates = dotT(inp, wihs[l]) + bihs[l] + dotT(hs[l], whhs[l]) + bhhs[l]
            gi = gates[:, 0:HID]
            gf = gates[:, HID:2 * HID]
            gg = gates[:, 2 * HID:3 * HID]
            go = gates[:, 3 * HID:4 * HID]
            cnew = jax.nn.sigmoid(gf) * cs[l] + jax.nn.sigmoid(gi) * jnp.tanh(gg)
            hnew = jax.nn.sigmoid(go) * jnp.tanh(cnew)
            hs[l] = hnew
            cs[l] = cnew
            inp = hnew
        q = inp
        e = dotT(feat, q)
        m = jnp.max(e)
        z = jnp.exp(e - m)
        ssum = jnp.sum(z)
        r = lax.dot_general(z, feat, (((0,), (0,)), ((), ())),
                            preferred_element_type=jnp.float32) / ssum
        q_star = jnp.concatenate([q, r], axis=1)
    out_ref[...] = q_star


def _k5(acc2, bias2, lstm):
    return pl.pallas_call(
        _k5_body,
        out_shape=jax.ShapeDtypeStruct((1, 2 * HID), jnp.float32),
    )(acc2, bias2, *lstm)


def kernel(x, edge_index, W_src1, W_dst1, attn1, bias1, W_src2, W_dst2,
           attn2, bias2, W_ih0, W_hh0, b_ih0, b_hh0, W_ih1, W_hh1, b_ih1,
           b_hh1, W_ih2, W_hh2, b_ih2, b_hh2):
    src = edge_index[0]
    dst = edge_index[1]
    attn1_t = attn1.reshape(NC, 128)
    bias1_r = bias1.reshape(1, HEADS * HID)
    bias2_r = bias2.reshape(1, HID)

    fs_t, fd_t = _k1(x, W_src1, W_dst1)
    acc1, den1 = _k2(fs_t, fd_t, src, dst, attn1_t)
    den1_t = den1.transpose(3, 0, 2, 1).reshape(NR, NC * 2 * NS)
    ft2, = _k3(acc1, den1_t, bias1_r, W_src2, W_dst2)
    acc2 = _k4(ft2, src, dst, attn2)
    lstm = (W_ih0, W_hh0, b_ih0.reshape(1, -1), b_hh0.reshape(1, -1),
            W_ih1, W_hh1, b_ih1.reshape(1, -1), b_hh1.reshape(1, -1),
            W_ih2, W_hh2, b_ih2.reshape(1, -1), b_hh2.reshape(1, -1))
    return _k5(acc2, bias2_r, lstm)


# speedup vs baseline: 1.5297x; 1.0572x over previous
import functools

import jax
import jax.numpy as jnp
from jax import lax
from jax.experimental import pallas as pl
from jax.experimental.pallas import tpu as pltpu
from jax.experimental.pallas import tpu_sc as plsc

N = 10000
E = 160000
IN_FEAT = 128
HID = 64
HEADS = 4
NEG_SLOPE = 0.2

NC = 2
NS = 16
L = 16

NR = 10240
DUMP = N
RPT = NR // NS

ET1 = E // NS
EC1 = 48
NP1 = 104
TAILE = ET1 - NP1 * 2 * EC1
EC = 64
ET2 = E // (NC * NS)
NF2 = ET2 // EC
TAIL2 = ET2 - NF2 * EC

def _mesh():
    return plsc.VectorSubcoreMesh(core_axis_name="c", subcore_axis_name="s",
                                  num_cores=NC, num_subcores=NS)


def _k1_body(x_ref, ws_ref, wd_ref, fs_ref, fd_ref):
    x = x_ref[...]
    fs_ref[...] = jnp.dot(x, ws_ref[...], preferred_element_type=jnp.float32)
    fd_ref[...] = jnp.dot(x, wd_ref[...], preferred_element_type=jnp.float32)


def _k1(x, W_src1, W_dst1):
    nb = 10
    rb = N // nb
    return pl.pallas_call(
        _k1_body,
        grid=(NC, nb),
        in_specs=[
            pl.BlockSpec((rb, IN_FEAT), lambda c, j: (j, 0)),
            pl.BlockSpec((IN_FEAT, 128), lambda c, j: (0, c)),
            pl.BlockSpec((IN_FEAT, 128), lambda c, j: (0, c)),
        ],
        out_specs=[
            pl.BlockSpec((rb, 128), lambda c, j: (c * nb + j, 0)),
            pl.BlockSpec((rb, 128), lambda c, j: (c * nb + j, 0)),
        ],
        out_shape=[
            jax.ShapeDtypeStruct((NC * N, 128), jnp.float32),
            jax.ShapeDtypeStruct((NC * N, 128), jnp.float32),
        ],
    )(x, W_src1, W_dst1)


def _k2_body(fs_hbm, fd_hbm, src_hbm, dst_hbm, attn_hbm,
             out_hbm, outden_hbm,
             srcA, dofA, drwA, srcB, dofB, drwB,
             fsA, fdA, fsB, fdB, attn_v,
             den0_v, den1_v, acc_s,
             smA1, smA2, smB1, smB2, sgA1, sgA2, sgB1, sgB2, ssA, ssB):
    c = lax.axis_index("c")
    s = lax.axis_index("s")
    zf = jnp.zeros((L,), jnp.float32)
    lane = lax.iota(jnp.int32, L)

    def _zrow(r, carry):
        for k in range(128 // L):
            fsA[r, pl.ds(k * L, L)] = zf
        return carry
    lax.fori_loop(0, EC1, _zrow, 0)
    base = s * RPT
    for blk in range(RPT // EC1):
        pltpu.sync_copy(fsA, acc_s.at[pl.ds(base + blk * EC1, EC1)])
    remz = RPT - (RPT // EC1) * EC1
    pltpu.sync_copy(fsA.at[pl.ds(0, remz)],
                    acc_s.at[pl.ds(base + (RPT // EC1) * EC1, remz)])

    def _zden(i, carry):
        den0_v[pl.ds(i * L, L)] = zf
        den1_v[pl.ds(i * L, L)] = zf
        return carry
    lax.fori_loop(0, NR // L, _zden, 0)

    pltpu.sync_copy(attn_hbm.at[c], attn_v)
    a_vecs = [attn_v[pl.ds(k * L, L)] for k in range(8)]
    an_vecs = [a * NEG_SLOPE for a in a_vecs]
    masks = [lane == i for i in range(L)]

    coff = jnp.full((L,), c * N, jnp.int32)
    ebase = s * ET1

    plsc.subcore_barrier()

    def _offsets(src_ch, dof_ch, drw_ch):
        for g in range(EC1 // L):
            src_ch[pl.ds(g * L, L)] = src_ch[pl.ds(g * L, L)] + coff
            dof_ch[pl.ds(g * L, L)] = drw_ch[pl.ds(g * L, L)] + coff

    def _compute(fs_rows, fd_rows, drw_ch, ssc):
        def _group(g, carry):
            dvec = drw_ch[pl.ds(g * L, L)]
            for e16 in range(L):
                e = g * L + e16
                fsv = [fs_rows[e, pl.ds(k * L, L)] for k in range(8)]
                fdv = [fd_rows[e, pl.ds(k * L, L)] for k in range(8)]
                av = []
                for k in range(8):
                    ev = fsv[k] + fdv[k]
                    av.append(ev * jnp.where(ev > 0, a_vecs[k], an_vecs[k]))
                s0 = jnp.sum((av[0] + av[1]) + (av[2] + av[3]))
                s1 = jnp.sum((av[4] + av[5]) + (av[6] + av[7]))
                ex0 = jnp.exp(jnp.full((L,), s0, jnp.float32))
                ex1 = jnp.exp(jnp.full((L,), s1, jnp.float32))
                for k in range(4):
                    fs_rows[e, pl.ds(k * L, L)] = fsv[k] * ex0
                for k in range(4, 8):
                    fs_rows[e, pl.ds(k * L, L)] = fsv[k] * ex1
                plsc.addupdate_scatter(den0_v, [dvec], ex0, mask=masks[e16])
                plsc.addupdate_scatter(den1_v, [dvec], ex1, mask=masks[e16])
            return carry
        lax.fori_loop(0, EC1 // L, _group, 0)
        return pltpu.async_copy(fs_rows, acc_s.at[drw_ch], ssc, add=True)

    pltpu.async_copy(src_hbm.at[pl.ds(ebase, EC1)], srcA, smA1)
    pltpu.async_copy(dst_hbm.at[pl.ds(ebase, EC1)], drwA, smA2)
    pltpu.async_copy(src_hbm.at[pl.ds(ebase + EC1, EC1)], srcB, smB1)
    pltpu.async_copy(dst_hbm.at[pl.ds(ebase + EC1, EC1)], drwB, smB2)

    def _pair(p, carry):
        ebA = ebase + (2 * p) * EC1
        pltpu.make_async_copy(src_hbm.at[pl.ds(ebA, EC1)], srcA, smA1).wait()
        pltpu.make_async_copy(dst_hbm.at[pl.ds(ebA, EC1)], drwA, smA2).wait()
        _offsets(srcA, dofA, drwA)
        gA1 = pltpu.async_copy(fs_hbm.at[srcA], fsA, sgA1)
        gA2 = pltpu.async_copy(fd_hbm.at[dofA], fdA, sgA2)
        pltpu.make_async_copy(src_hbm.at[pl.ds(ebA, EC1)], srcB, smB1).wait()
        pltpu.make_async_copy(dst_hbm.at[pl.ds(ebA, EC1)], drwB, smB2).wait()
        _offsets(srcB, dofB, drwB)
        gB1 = pltpu.async_copy(fs_hbm.at[srcB], fsB, sgB1)
        gB2 = pltpu.async_copy(fd_hbm.at[dofB], fdB, sgB2)
        gA1.wait()
        gA2.wait()
        scA = _compute(fsA, fdA, drwA, ssA)
        gB1.wait()
        gB2.wait()
        scB = _compute(fsB, fdB, drwB, ssB)
        ebA2 = jnp.minimum(ebA + 2 * EC1, NS * ET1 - 2 * EC1)
        scA.wait()
        pltpu.async_copy(src_hbm.at[pl.ds(ebA2, EC1)], srcA, smA1)
        pltpu.async_copy(dst_hbm.at[pl.ds(ebA2, EC1)], drwA, smA2)
        scB.wait()
        pltpu.async_copy(src_hbm.at[pl.ds(ebA2 + EC1, EC1)], srcB, smB1)
        pltpu.async_copy(dst_hbm.at[pl.ds(ebA2 + EC1, EC1)], drwB, smB2)
        return carry
    lax.fori_loop(0, NP1, _pair, 0)
    pltpu.make_async_copy(src_hbm.at[pl.ds(0, EC1)], srcA, smA1).wait()
    pltpu.make_async_copy(dst_hbm.at[pl.ds(0, EC1)], drwA, smA2).wait()
    pltpu.make_async_copy(src_hbm.at[pl.ds(0, EC1)], srcB, smB1).wait()
    pltpu.make_async_copy(dst_hbm.at[pl.ds(0, EC1)], drwB, smB2).wait()

    pltpu.sync_copy(src_hbm.at[pl.ds(ebase + NP1 * 2 * EC1, TAILE)],
                    srcA.at[pl.ds(0, TAILE)])
    pltpu.sync_copy(dst_hbm.at[pl.ds(ebase + NP1 * 2 * EC1, TAILE)],
                    drwA.at[pl.ds(0, TAILE)])
    for t in range(TAILE // L, EC1 // L):
        srcA[pl.ds(t * L, L)] = jnp.full((L,), 0, jnp.int32)
        drwA[pl.ds(t * L, L)] = jnp.full((L,), DUMP, jnp.int32)
    _offsets(srcA, dofA, drwA)
    tA1 = pltpu.async_copy(fs_hbm.at[srcA], fsA, sgA1)
    tA2 = pltpu.async_copy(fd_hbm.at[dofA], fdA, sgA2)
    tA1.wait()
    tA2.wait()
    _compute(fsA, fdA, drwA, ssA).wait()

    pltpu.sync_copy(den0_v, outden_hbm.at[c, s, 0])
    pltpu.sync_copy(den1_v, outden_hbm.at[c, s, 1])

    plsc.subcore_barrier()
    pltpu.sync_copy(acc_s.at[pl.ds(base, RPT)], out_hbm.at[c, pl.ds(base, RPT)])


def _k2(fs_t, fd_t, src, dst, attn_t):
    f = functools.partial(
        pl.kernel,
        out_type=(
            jax.ShapeDtypeStruct((NC, NR, 128), jnp.float32),
            jax.ShapeDtypeStruct((NC, NS, 2, NR), jnp.float32),
        ),
        mesh=_mesh(),
        compiler_params=pltpu.CompilerParams(needs_layout_passes=False),
        scratch_types=[
            pltpu.VMEM((EC1,), jnp.int32),
            pltpu.VMEM((EC1,), jnp.int32),
            pltpu.VMEM((EC1,), jnp.int32),
            pltpu.VMEM((EC1,), jnp.int32),
            pltpu.VMEM((EC1,), jnp.int32),
            pltpu.VMEM((EC1,), jnp.int32),
            pltpu.VMEM((EC1, 128), jnp.float32),
            pltpu.VMEM((EC1, 128), jnp.float32),
            pltpu.VMEM((EC1, 128), jnp.float32),
            pltpu.VMEM((EC1, 128), jnp.float32),
            pltpu.VMEM((128,), jnp.float32),
            pltpu.VMEM((NR,), jnp.float32),
            pltpu.VMEM((NR,), jnp.float32),
            pltpu.VMEM_SHARED((NR, 128), jnp.float32),
            pltpu.SemaphoreType.DMA,
            pltpu.SemaphoreType.DMA,
            pltpu.SemaphoreType.DMA,
            pltpu.SemaphoreType.DMA,
            pltpu.SemaphoreType.DMA,
            pltpu.SemaphoreType.DMA,
            pltpu.SemaphoreType.DMA,
            pltpu.SemaphoreType.DMA,
            pltpu.SemaphoreType.DMA,
            pltpu.SemaphoreType.DMA,
        ],
    )(_k2_body)
    return f(fs_t, fd_t, src, dst, attn_t)


def _k3_body(acc_ref, den_ref, b_ref, ws_ref, wd_ref, ft_ref):
    a = acc_ref[...]
    d = den_ref[...]
    parts = []
    for c in range(NC):
        for k in range(2):
            num = a[c, :, k * 64:(k + 1) * 64]
            den = jnp.sum(d[:, (c * 2 + k) * NS:(c * 2 + k + 1) * NS],
                          axis=1, keepdims=True)
            parts.append(num / jnp.maximum(den, 1e-16))
    h = jnp.concatenate(parts, axis=1) + b_ref[...]
    h = jnp.where(h > 0, h, jnp.exp(h) - 1.0)
    ft_ref[...] = jnp.concatenate(
        [jnp.dot(h, ws_ref[...], preferred_element_type=jnp.float32),
         jnp.dot(h, wd_ref[...], preferred_element_type=jnp.float32)], axis=1)


def _k3(acc1, den1, bias1, W_src2, W_dst2):
    nb = 10
    rb = N // nb
    return pl.pallas_call(
        _k3_body,
        grid=(nb,),
        in_specs=[
            pl.BlockSpec((NC, rb, 128), lambda j: (0, j, 0)),
            pl.BlockSpec((rb, NC * 2 * NS), lambda j: (j, 0)),
            pl.BlockSpec((1, HEADS * HID), lambda j: (0, 0)),
            pl.BlockSpec((HEADS * HID, HID), lambda j: (0, 0)),
            pl.BlockSpec((HEADS * HID, HID), lambda j: (0, 0)),
        ],
        out_specs=[
            pl.BlockSpec((rb, 2 * HID), lambda j: (j, 0)),
        ],
        out_shape=[
            jax.ShapeDtypeStruct((N, 2 * HID), jnp.float32),
        ],
    )(acc1, den1, bias1, W_src2, W_dst2)


def _k4_body(ft_hbm, src_hbm, dst_hbm, attn_hbm, out_hbm,
             srcA, dstA, srcB, dstB, fsA, fdA, fsB, fdB, attn_v,
             acc_s, smA1, smA2, smB1, smB2, sgA1, sgA2, sgB1, sgB2):
    c = lax.axis_index("c")
    s = lax.axis_index("s")
    zf = jnp.zeros((L,), jnp.float32)
    lane = lax.iota(jnp.int32, L)

    def _zrow(r, carry):
        for k in range(128 // L):
            fsA[r, pl.ds(k * L, L)] = zf
        return carry
    lax.fori_loop(0, EC, _zrow, 0)
    base = s * RPT
    for blk in range(RPT // EC):
        pltpu.sync_copy(fsA, acc_s.at[pl.ds(base + blk * EC, EC)])

    pltpu.sync_copy(attn_hbm.at[0], attn_v)
    a_vecs = [attn_v[pl.ds(k * L, L)] for k in range(4)]
    an_vecs = [a * NEG_SLOPE for a in a_vecs]

    ebase = (c * NS + s) * ET2

    plsc.subcore_barrier()

    def _compute(fs_rows, fd_rows, dst_ch):
        def _edge(e, carry):
            fsv = [fs_rows[e, pl.ds(k * L, L)] for k in range(4)]
            fdv = [fd_rows[e, pl.ds((4 + k) * L, L)] for k in range(4)]
            av = []
            for k in range(4):
                ev = fsv[k] + fdv[k]
                av.append(ev * jnp.where(ev > 0, a_vecs[k], an_vecs[k]))
            s0 = jnp.sum((av[0] + av[1]) + (av[2] + av[3]))
            ex0 = jnp.exp(jnp.full((L,), s0, jnp.float32))
            for k in range(4):
                fs_rows[e, pl.ds(k * L, L)] = fsv[k] * ex0
            fs_rows[e, pl.ds(64, L)] = jnp.where(lane == 0, ex0, zf)
            for k in range(5, 8):
                fs_rows[e, pl.ds(k * L, L)] = zf
            return carry
        lax.fori_loop(0, EC, _edge, 0)
        pltpu.sync_copy(fs_rows, acc_s.at[dst_ch], add=True)

    def _pair(p, carry):
        ebA = ebase + (2 * p) * EC
        ebB = ebA + EC
        iA1 = pltpu.async_copy(src_hbm.at[pl.ds(ebA, EC)], srcA, smA1)
        iA2 = pltpu.async_copy(dst_hbm.at[pl.ds(ebA, EC)], dstA, smA2)
        iB1 = pltpu.async_copy(src_hbm.at[pl.ds(ebB, EC)], srcB, smB1)
        iB2 = pltpu.async_copy(dst_hbm.at[pl.ds(ebB, EC)], dstB, smB2)
        iA1.wait()
        iA2.wait()
        gA1 = pltpu.async_copy(ft_hbm.at[srcA], fsA, sgA1)
        gA2 = pltpu.async_copy(ft_hbm.at[dstA], fdA, sgA2)
        iB1.wait()
        iB2.wait()
        gB1 = pltpu.async_copy(ft_hbm.at[srcB], fsB, sgB1)
        gB2 = pltpu.async_copy(ft_hbm.at[dstB], fdB, sgB2)
        gA1.wait()
        gA2.wait()
        _compute(fsA, fdA, dstA)
        gB1.wait()
        gB2.wait()
        _compute(fsB, fdB, dstB)
        return carry
    lax.fori_loop(0, NF2 // 2, _pair, 0)

    pltpu.sync_copy(src_hbm.at[pl.ds(ebase + NF2 * EC, TAIL2)],
                    srcA.at[pl.ds(0, TAIL2)])
    pltpu.sync_copy(dst_hbm.at[pl.ds(ebase + NF2 * EC, TAIL2)],
                    dstA.at[pl.ds(0, TAIL2)])
    tail_keep = lane < TAIL2
    srcA[pl.ds(0, L)] = jnp.where(tail_keep, srcA[pl.ds(0, L)], 0)
    dstA[pl.ds(0, L)] = jnp.where(tail_keep, dstA[pl.ds(0, L)], DUMP)
    for t in range(1, EC // L):
        srcA[pl.ds(t * L, L)] = jnp.full((L,), 0, jnp.int32)
        dstA[pl.ds(t * L, L)] = jnp.full((L,), DUMP, jnp.int32)
    tA1 = pltpu.async_copy(ft_hbm.at[srcA], fsA, sgA1)
    tA2 = pltpu.async_copy(ft_hbm.at[dstA], fdA, sgA2)
    tA1.wait()
    tA2.wait()
    _compute(fsA, fdA, dstA)

    plsc.subcore_barrier()
    pltpu.sync_copy(acc_s.at[pl.ds(base, RPT)], out_hbm.at[c, pl.ds(base, RPT)])


def _k4(ft2, src, dst, attn2):
    f = functools.partial(
        pl.kernel,
        out_type=jax.ShapeDtypeStruct((NC, NR, 128), jnp.float32),
        mesh=_mesh(),
        compiler_params=pltpu.CompilerParams(needs_layout_passes=False),
        scratch_types=[
            pltpu.VMEM((EC,), jnp.int32),
            pltpu.VMEM((EC,), jnp.int32),
            pltpu.VMEM((EC,), jnp.int32),
            pltpu.VMEM((EC,), jnp.int32),
            pltpu.VMEM((EC, 128), jnp.float32),
            pltpu.VMEM((EC, 128), jnp.float32),
            pltpu.VMEM((EC, 128), jnp.float32),
            pltpu.VMEM((EC, 128), jnp.float32),
            pltpu.VMEM((HID,), jnp.float32),
            pltpu.VMEM_SHARED((NR, 128), jnp.float32),
            pltpu.SemaphoreType.DMA,
            pltpu.SemaphoreType.DMA,
            pltpu.SemaphoreType.DMA,
            pltpu.SemaphoreType.DMA,
            pltpu.SemaphoreType.DMA,
            pltpu.SemaphoreType.DMA,
            pltpu.SemaphoreType.DMA,
            pltpu.SemaphoreType.DMA,
        ],
    )(_k4_body)
    return f(ft2, src, dst, attn2)


def _k5_body(acc_ref, b2_ref,
             wih0, whh0, bih0, bhh0, wih1, whh1, bih1, bhh1,
             wih2, whh2, bih2, bhh2, out_ref):
    a = acc_ref[...]
    num = a[0, :N, :HID] + a[1, :N, :HID]
    den = a[0, :N, HID:HID + 1] + a[1, :N, HID:HID + 1]
    h = num / jnp.maximum(den, 1e-16) + b2_ref[...]
    feat = jnp.where(h > 0, h, jnp.exp(h) - 1.0)

    wihs = (wih0[...], wih1[...], wih2[...])
    whhs = (whh0[...], whh1[...], whh2[...])
    bihs = (bih0[...], bih1[...], bih2[...])
    bhhs = (bhh0[...], bhh1[...], bhh2[...])

    hs = [jnp.zeros((1, HID), jnp.float32) for _ in range(3)]
    cs = [jnp.zeros((1, HID), jnp.float32) for _ in range(3)]
    q_star = jnp.zeros((1, 2 * HID), jnp.float32)

    def dotT(u, w):
        return lax.dot_general(u, w, (((1,), (1,)), ((), ())),
                               preferred_element_type=jnp.float32)

    for _ in range(3):
        inp = q_star
        for l in range(3):
            gates = dotT(inp, wihs[l]) + bihs[l] + dotT(hs[l], whhs[l]) + bhhs[l]
            gi = gates[:, 0:HID]
            gf = gates[:, HID:2 * HID]
            gg = gates[:, 2 * HID:3 * HID]
            go = gates[:, 3 * HID:4 * HID]
            cnew = jax.nn.sigmoid(gf) * cs[l] + jax.nn.sigmoid(gi) * jnp.tanh(gg)
            hnew = jax.nn.sigmoid(go) * jnp.tanh(cnew)
            hs[l] = hnew
            cs[l] = cnew
            inp = hnew
        q = inp
        e = dotT(feat, q)
        m = jnp.max(e)
        z = jnp.exp(e - m)
        ssum = jnp.sum(z)
        r = lax.dot_general(z, feat, (((0,), (0,)), ((), ())),
                            preferred_element_type=jnp.float32) / ssum
        q_star = jnp.concatenate([q, r], axis=1)
    out_ref[...] = q_star


def _k5(acc2, bias2, lstm):
    return pl.pallas_call(
        _k5_body,
        out_shape=jax.ShapeDtypeStruct((1, 2 * HID), jnp.float32),
    )(acc2, bias2, *lstm)


def kernel(x, edge_index, W_src1, W_dst1, attn1, bias1, W_src2, W_dst2,
           attn2, bias2, W_ih0, W_hh0, b_ih0, b_hh0, W_ih1, W_hh1, b_ih1,
           b_hh1, W_ih2, W_hh2, b_ih2, b_hh2):
    src = edge_index[0]
    dst = edge_index[1]
    attn1_t = attn1.reshape(NC, 128)
    bias1_r = bias1.reshape(1, HEADS * HID)
    bias2_r = bias2.reshape(1, HID)

    fs_t, fd_t = _k1(x, W_src1, W_dst1)
    acc1, den1 = _k2(fs_t, fd_t, src, dst, attn1_t)
    den1_t = den1.transpose(3, 0, 2, 1).reshape(NR, NC * 2 * NS)
    ft2, = _k3(acc1, den1_t, bias1_r, W_src2, W_dst2)
    acc2 = _k4(ft2, src, dst, attn2)
    lstm = (W_ih0, W_hh0, b_ih0.reshape(1, -1), b_hh0.reshape(1, -1),
            W_ih1, W_hh1, b_ih1.reshape(1, -1), b_hh1.reshape(1, -1),
            W_ih2, W_hh2, b_ih2.reshape(1, -1), b_hh2.reshape(1, -1))
    return _k5(acc2, bias2_r, lstm)


# speedup vs baseline: 1.5803x; 1.0331x over previous
import functools

import jax
import jax.numpy as jnp
from jax import lax
from jax.experimental import pallas as pl
from jax.experimental.pallas import tpu as pltpu
from jax.experimental.pallas import tpu_sc as plsc

N = 10000
E = 160000
IN_FEAT = 128
HID = 64
HEADS = 4
NEG_SLOPE = 0.2

NC = 2
NS = 16
L = 16

NR = 10240
DUMP = N
RPT = NR // NS

ET1 = E // NS
EC1 = 48
NP1 = 104
TAILE = ET1 - NP1 * 2 * EC1
EC = 64
ET2 = E // (NC * NS)
NF2 = ET2 // EC
TAIL2 = ET2 - NF2 * EC

def _mesh():
    return plsc.VectorSubcoreMesh(core_axis_name="c", subcore_axis_name="s",
                                  num_cores=NC, num_subcores=NS)


def _k1_body(x_ref, ws_ref, wd_ref, fs_ref, fd_ref):
    x = x_ref[...]
    fs_ref[...] = jnp.dot(x, ws_ref[...], preferred_element_type=jnp.float32)
    fd_ref[...] = jnp.dot(x, wd_ref[...], preferred_element_type=jnp.float32)


def _k1(x, W_src1, W_dst1):
    nb = 10
    rb = N // nb
    return pl.pallas_call(
        _k1_body,
        grid=(NC, nb),
        in_specs=[
            pl.BlockSpec((rb, IN_FEAT), lambda c, j: (j, 0)),
            pl.BlockSpec((IN_FEAT, 128), lambda c, j: (0, c)),
            pl.BlockSpec((IN_FEAT, 128), lambda c, j: (0, c)),
        ],
        out_specs=[
            pl.BlockSpec((rb, 128), lambda c, j: (c * nb + j, 0)),
            pl.BlockSpec((rb, 128), lambda c, j: (c * nb + j, 0)),
        ],
        out_shape=[
            jax.ShapeDtypeStruct((NC * N, 128), jnp.float32),
            jax.ShapeDtypeStruct((NC * N, 128), jnp.float32),
        ],
    )(x, W_src1, W_dst1)


def _k2_body(fs_hbm, fd_hbm, src_hbm, dst_hbm, attn_hbm,
             out_hbm, outden_hbm,
             srcA, dofA, drwA, srcB, dofB, drwB,
             fsA, fdA, fsB, fdB, attn_v,
             den0_v, den1_v, acc_s,
             smA1, smA2, smB1, smB2, sgA1, sgA2, sgB1, sgB2, ssA, ssB):
    c = lax.axis_index("c")
    s = lax.axis_index("s")
    zf = jnp.zeros((L,), jnp.float32)
    lane = lax.iota(jnp.int32, L)

    def _zrow(r, carry):
        for k in range(128 // L):
            fsA[r, pl.ds(k * L, L)] = zf
        return carry
    lax.fori_loop(0, EC1, _zrow, 0)
    base = s * RPT
    for blk in range(RPT // EC1):
        pltpu.sync_copy(fsA, acc_s.at[pl.ds(base + blk * EC1, EC1)])
    remz = RPT - (RPT // EC1) * EC1
    pltpu.sync_copy(fsA.at[pl.ds(0, remz)],
                    acc_s.at[pl.ds(base + (RPT // EC1) * EC1, remz)])

    def _zden(i, carry):
        den0_v[pl.ds(i * L, L)] = zf
        den1_v[pl.ds(i * L, L)] = zf
        return carry
    lax.fori_loop(0, NR // L, _zden, 0)

    pltpu.sync_copy(attn_hbm.at[c], attn_v)
    a_vecs = [attn_v[pl.ds(k * L, L)] for k in range(8)]
    an_vecs = [a * NEG_SLOPE for a in a_vecs]
    masks = [lane == i for i in range(L)]

    coff = jnp.full((L,), c * N, jnp.int32)
    ebase = s * ET1

    plsc.subcore_barrier()

    def _offsets(src_ch, dof_ch, drw_ch):
        for g in range(EC1 // L):
            src_ch[pl.ds(g * L, L)] = src_ch[pl.ds(g * L, L)] + coff
            dof_ch[pl.ds(g * L, L)] = drw_ch[pl.ds(g * L, L)] + coff

    def _compute(fs_rows, fd_rows, drw_ch, ssc):
        def _group(g, carry):
            dvec = drw_ch[pl.ds(g * L, L)]
            for e16 in range(L):
                e = g * L + e16
                fsv = [fs_rows[e, pl.ds(k * L, L)] for k in range(8)]
                fdv = [fd_rows[e, pl.ds(k * L, L)] for k in range(8)]
                av = []
                for k in range(8):
                    ev = fsv[k] + fdv[k]
                    av.append(ev * jnp.where(ev > 0, a_vecs[k], an_vecs[k]))
                s0 = jnp.sum((av[0] + av[1]) + (av[2] + av[3]))
                s1 = jnp.sum((av[4] + av[5]) + (av[6] + av[7]))
                ex0 = jnp.exp(jnp.full((L,), s0, jnp.float32))
                ex1 = jnp.exp(jnp.full((L,), s1, jnp.float32))
                for k in range(4):
                    fs_rows[e, pl.ds(k * L, L)] = fsv[k] * ex0
                for k in range(4, 8):
                    fs_rows[e, pl.ds(k * L, L)] = fsv[k] * ex1
                plsc.addupdate_scatter(den0_v, [dvec], ex0, mask=masks[e16])
                plsc.addupdate_scatter(den1_v, [dvec], ex1, mask=masks[e16])
            return carry
        lax.fori_loop(0, EC1 // L, _group, 0)
        return pltpu.async_copy(fs_rows, acc_s.at[drw_ch], ssc, add=True)

    pltpu.async_copy(src_hbm.at[pl.ds(ebase, EC1)], srcA, smA1)
    pltpu.async_copy(dst_hbm.at[pl.ds(ebase, EC1)], drwA, smA2)
    pltpu.async_copy(src_hbm.at[pl.ds(ebase + EC1, EC1)], srcB, smB1)
    pltpu.async_copy(dst_hbm.at[pl.ds(ebase + EC1, EC1)], drwB, smB2)

    def _pair(p, carry):
        ebA = ebase + (2 * p) * EC1
        pltpu.make_async_copy(src_hbm.at[pl.ds(ebA, EC1)], srcA, smA1).wait()
        pltpu.make_async_copy(dst_hbm.at[pl.ds(ebA, EC1)], drwA, smA2).wait()
        _offsets(srcA, dofA, drwA)
        gA1 = pltpu.async_copy(fs_hbm.at[srcA], fsA, sgA1)
        gA2 = pltpu.async_copy(fd_hbm.at[dofA], fdA, sgA2)
        pltpu.make_async_copy(src_hbm.at[pl.ds(ebA, EC1)], srcB, smB1).wait()
        pltpu.make_async_copy(dst_hbm.at[pl.ds(ebA, EC1)], drwB, smB2).wait()
        _offsets(srcB, dofB, drwB)
        gB1 = pltpu.async_copy(fs_hbm.at[srcB], fsB, sgB1)
        gB2 = pltpu.async_copy(fd_hbm.at[dofB], fdB, sgB2)
        gA1.wait()
        gA2.wait()
        scA = _compute(fsA, fdA, drwA, ssA)
        gB1.wait()
        gB2.wait()
        scB = _compute(fsB, fdB, drwB, ssB)
        ebA2 = jnp.minimum(ebA + 2 * EC1, NS * ET1 - 2 * EC1)
        scA.wait()
        pltpu.async_copy(src_hbm.at[pl.ds(ebA2, EC1)], srcA, smA1)
        pltpu.async_copy(dst_hbm.at[pl.ds(ebA2, EC1)], drwA, smA2)
        scB.wait()
        pltpu.async_copy(src_hbm.at[pl.ds(ebA2 + EC1, EC1)], srcB, smB1)
        pltpu.async_copy(dst_hbm.at[pl.ds(ebA2 + EC1, EC1)], drwB, smB2)
        return carry
    lax.fori_loop(0, NP1, _pair, 0)
    pltpu.make_async_copy(src_hbm.at[pl.ds(0, EC1)], srcA, smA1).wait()
    pltpu.make_async_copy(dst_hbm.at[pl.ds(0, EC1)], drwA, smA2).wait()
    pltpu.make_async_copy(src_hbm.at[pl.ds(0, EC1)], srcB, smB1).wait()
    pltpu.make_async_copy(dst_hbm.at[pl.ds(0, EC1)], drwB, smB2).wait()

    pltpu.sync_copy(src_hbm.at[pl.ds(ebase + NP1 * 2 * EC1, TAILE)],
                    srcA.at[pl.ds(0, TAILE)])
    pltpu.sync_copy(dst_hbm.at[pl.ds(ebase + NP1 * 2 * EC1, TAILE)],
                    drwA.at[pl.ds(0, TAILE)])
    for t in range(TAILE // L, EC1 // L):
        srcA[pl.ds(t * L, L)] = jnp.full((L,), 0, jnp.int32)
        drwA[pl.ds(t * L, L)] = jnp.full((L,), DUMP, jnp.int32)
    _offsets(srcA, dofA, drwA)
    tA1 = pltpu.async_copy(fs_hbm.at[srcA], fsA, sgA1)
    tA2 = pltpu.async_copy(fd_hbm.at[dofA], fdA, sgA2)
    tA1.wait()
    tA2.wait()
    _compute(fsA, fdA, drwA, ssA).wait()

    pltpu.sync_copy(den0_v, outden_hbm.at[c, s, 0])
    pltpu.sync_copy(den1_v, outden_hbm.at[c, s, 1])

    plsc.subcore_barrier()
    pltpu.sync_copy(acc_s.at[pl.ds(base, RPT)], out_hbm.at[c, pl.ds(base, RPT)])


def _k2(fs_t, fd_t, src, dst, attn_t):
    f = functools.partial(
        pl.kernel,
        out_type=(
            jax.ShapeDtypeStruct((NC, NR, 128), jnp.float32),
            jax.ShapeDtypeStruct((NC, NS, 2, NR), jnp.float32),
        ),
        mesh=_mesh(),
        compiler_params=pltpu.CompilerParams(needs_layout_passes=False),
        scratch_types=[
            pltpu.VMEM((EC1,), jnp.int32),
            pltpu.VMEM((EC1,), jnp.int32),
            pltpu.VMEM((EC1,), jnp.int32),
            pltpu.VMEM((EC1,), jnp.int32),
            pltpu.VMEM((EC1,), jnp.int32),
            pltpu.VMEM((EC1,), jnp.int32),
            pltpu.VMEM((EC1, 128), jnp.float32),
            pltpu.VMEM((EC1, 128), jnp.float32),
            pltpu.VMEM((EC1, 128), jnp.float32),
            pltpu.VMEM((EC1, 128), jnp.float32),
            pltpu.VMEM((128,), jnp.float32),
            pltpu.VMEM((NR,), jnp.float32),
            pltpu.VMEM((NR,), jnp.float32),
            pltpu.VMEM_SHARED((NR, 128), jnp.float32),
            pltpu.SemaphoreType.DMA,
            pltpu.SemaphoreType.DMA,
            pltpu.SemaphoreType.DMA,
            pltpu.SemaphoreType.DMA,
            pltpu.SemaphoreType.DMA,
            pltpu.SemaphoreType.DMA,
            pltpu.SemaphoreType.DMA,
            pltpu.SemaphoreType.DMA,
            pltpu.SemaphoreType.DMA,
            pltpu.SemaphoreType.DMA,
        ],
    )(_k2_body)
    return f(fs_t, fd_t, src, dst, attn_t)


def _k3_body(acc_ref, den_ref, b_ref, ws_ref, wd_ref, ft_ref):
    a = acc_ref[...]
    d = den_ref[...]
    parts = []
    for c in range(NC):
        for k in range(2):
            num = a[c, :, k * 64:(k + 1) * 64]
            den = jnp.sum(d[:, (c * 2 + k) * NS:(c * 2 + k + 1) * NS],
                          axis=1, keepdims=True)
            parts.append(num / jnp.maximum(den, 1e-16))
    h = jnp.concatenate(parts, axis=1) + b_ref[...]
    h = jnp.where(h > 0, h, jnp.exp(h) - 1.0)
    ft_ref[...] = jnp.concatenate(
        [jnp.dot(h, ws_ref[...], preferred_element_type=jnp.float32),
         jnp.dot(h, wd_ref[...], preferred_element_type=jnp.float32)], axis=1)


def _k3(acc1, den1, bias1, W_src2, W_dst2):
    nb = 10
    rb = N // nb
    return pl.pallas_call(
        _k3_body,
        grid=(nb,),
        in_specs=[
            pl.BlockSpec((NC, rb, 128), lambda j: (0, j, 0)),
            pl.BlockSpec((rb, NC * 2 * NS), lambda j: (j, 0)),
            pl.BlockSpec((1, HEADS * HID), lambda j: (0, 0)),
            pl.BlockSpec((HEADS * HID, HID), lambda j: (0, 0)),
            pl.BlockSpec((HEADS * HID, HID), lambda j: (0, 0)),
        ],
        out_specs=[
            pl.BlockSpec((rb, 2 * HID), lambda j: (j, 0)),
        ],
        out_shape=[
            jax.ShapeDtypeStruct((N, 2 * HID), jnp.float32),
        ],
    )(acc1, den1, bias1, W_src2, W_dst2)


def _k4_body(ft_hbm, src_hbm, dst_hbm, attn_hbm, out_hbm,
             srcA, dstA, srcB, dstB, fsA, fdA, fsB, fdB, attn_v,
             acc_s, smA1, smA2, smB1, smB2, sgA1, sgA2, sgB1, sgB2, ssA, ssB):
    c = lax.axis_index("c")
    s = lax.axis_index("s")
    zf = jnp.zeros((L,), jnp.float32)
    lane = lax.iota(jnp.int32, L)

    def _zrow(r, carry):
        for k in range(128 // L):
            fsA[r, pl.ds(k * L, L)] = zf
        return carry
    lax.fori_loop(0, EC, _zrow, 0)
    base = s * RPT
    for blk in range(RPT // EC):
        pltpu.sync_copy(fsA, acc_s.at[pl.ds(base + blk * EC, EC)])

    pltpu.sync_copy(attn_hbm.at[0], attn_v)
    a_vecs = [attn_v[pl.ds(k * L, L)] for k in range(4)]
    an_vecs = [a * NEG_SLOPE for a in a_vecs]

    ebase = (c * NS + s) * ET2

    plsc.subcore_barrier()

    def _compute(fs_rows, fd_rows, dst_ch, ssc):
        def _edge(e, carry):
            fsv = [fs_rows[e, pl.ds(k * L, L)] for k in range(4)]
            fdv = [fd_rows[e, pl.ds((4 + k) * L, L)] for k in range(4)]
            av = []
            for k in range(4):
                ev = fsv[k] + fdv[k]
                av.append(ev * jnp.where(ev > 0, a_vecs[k], an_vecs[k]))
            s0 = jnp.sum((av[0] + av[1]) + (av[2] + av[3]))
            ex0 = jnp.exp(jnp.full((L,), s0, jnp.float32))
            for k in range(4):
                fs_rows[e, pl.ds(k * L, L)] = fsv[k] * ex0
            fs_rows[e, pl.ds(64, L)] = jnp.where(lane == 0, ex0, zf)
            for k in range(5, 8):
                fs_rows[e, pl.ds(k * L, L)] = zf
            return carry
        lax.fori_loop(0, EC, _edge, 0)
        return pltpu.async_copy(fs_rows, acc_s.at[dst_ch], ssc, add=True)

    pltpu.async_copy(src_hbm.at[pl.ds(ebase, EC)], srcA, smA1)
    pltpu.async_copy(dst_hbm.at[pl.ds(ebase, EC)], dstA, smA2)
    pltpu.async_copy(src_hbm.at[pl.ds(ebase + EC, EC)], srcB, smB1)
    pltpu.async_copy(dst_hbm.at[pl.ds(ebase + EC, EC)], dstB, smB2)

    def _pair(p, carry):
        ebA = ebase + (2 * p) * EC
        pltpu.make_async_copy(src_hbm.at[pl.ds(ebA, EC)], srcA, smA1).wait()
        pltpu.make_async_copy(dst_hbm.at[pl.ds(ebA, EC)], dstA, smA2).wait()
        gA1 = pltpu.async_copy(ft_hbm.at[srcA], fsA, sgA1)
        gA2 = pltpu.async_copy(ft_hbm.at[dstA], fdA, sgA2)
        pltpu.make_async_copy(src_hbm.at[pl.ds(ebA, EC)], srcB, smB1).wait()
        pltpu.make_async_copy(dst_hbm.at[pl.ds(ebA, EC)], dstB, smB2).wait()
        gB1 = pltpu.async_copy(ft_hbm.at[srcB], fsB, sgB1)
        gB2 = pltpu.async_copy(ft_hbm.at[dstB], fdB, sgB2)
        gA1.wait()
        gA2.wait()
        scA = _compute(fsA, fdA, dstA, ssA)
        gB1.wait()
        gB2.wait()
        scB = _compute(fsB, fdB, dstB, ssB)
        ebA2 = jnp.minimum(ebA + 2 * EC, E - 2 * EC)
        scA.wait()
        pltpu.async_copy(src_hbm.at[pl.ds(ebA2, EC)], srcA, smA1)
        pltpu.async_copy(dst_hbm.at[pl.ds(ebA2, EC)], dstA, smA2)
        scB.wait()
        pltpu.async_copy(src_hbm.at[pl.ds(ebA2 + EC, EC)], srcB, smB1)
        pltpu.async_copy(dst_hbm.at[pl.ds(ebA2 + EC, EC)], dstB, smB2)
        return carry
    lax.fori_loop(0, NF2 // 2, _pair, 0)
    pltpu.make_async_copy(src_hbm.at[pl.ds(0, EC)], srcA, smA1).wait()
    pltpu.make_async_copy(dst_hbm.at[pl.ds(0, EC)], dstA, smA2).wait()
    pltpu.make_async_copy(src_hbm.at[pl.ds(0, EC)], srcB, smB1).wait()
    pltpu.make_async_copy(dst_hbm.at[pl.ds(0, EC)], dstB, smB2).wait()

    pltpu.sync_copy(src_hbm.at[pl.ds(ebase + NF2 * EC, TAIL2)],
                    srcA.at[pl.ds(0, TAIL2)])
    pltpu.sync_copy(dst_hbm.at[pl.ds(ebase + NF2 * EC, TAIL2)],
                    dstA.at[pl.ds(0, TAIL2)])
    tail_keep = lane < TAIL2
    srcA[pl.ds(0, L)] = jnp.where(tail_keep, srcA[pl.ds(0, L)], 0)
    dstA[pl.ds(0, L)] = jnp.where(tail_keep, dstA[pl.ds(0, L)], DUMP)
    for t in range(1, EC // L):
        srcA[pl.ds(t * L, L)] = jnp.full((L,), 0, jnp.int32)
        dstA[pl.ds(t * L, L)] = jnp.full((L,), DUMP, jnp.int32)
    tA1 = pltpu.async_copy(ft_hbm.at[srcA], fsA, sgA1)
    tA2 = pltpu.async_copy(ft_hbm.at[dstA], fdA, sgA2)
    tA1.wait()
    tA2.wait()
    _compute(fsA, fdA, dstA, ssA).wait()

    plsc.subcore_barrier()
    pltpu.sync_copy(acc_s.at[pl.ds(base, RPT)], out_hbm.at[c, pl.ds(base, RPT)])


def _k4(ft2, src, dst, attn2):
    f = functools.partial(
        pl.kernel,
        out_type=jax.ShapeDtypeStruct((NC, NR, 128), jnp.float32),
        mesh=_mesh(),
        compiler_params=pltpu.CompilerParams(needs_layout_passes=False),
        scratch_types=[
            pltpu.VMEM((EC,), jnp.int32),
            pltpu.VMEM((EC,), jnp.int32),
            pltpu.VMEM((EC,), jnp.int32),
            pltpu.VMEM((EC,), jnp.int32),
            pltpu.VMEM((EC, 128), jnp.float32),
            pltpu.VMEM((EC, 128), jnp.float32),
            pltpu.VMEM((EC, 128), jnp.float32),
            pltpu.VMEM((EC, 128), jnp.float32),
            pltpu.VMEM((HID,), jnp.float32),
            pltpu.VMEM_SHARED((NR, 128), jnp.float32),
            pltpu.SemaphoreType.DMA,
            pltpu.SemaphoreType.DMA,
            pltpu.SemaphoreType.DMA,
            pltpu.SemaphoreType.DMA,
            pltpu.SemaphoreType.DMA,
            pltpu.SemaphoreType.DMA,
            pltpu.SemaphoreType.DMA,
            pltpu.SemaphoreType.DMA,
            pltpu.SemaphoreType.DMA,
            pltpu.SemaphoreType.DMA,
        ],
    )(_k4_body)
    return f(ft2, src, dst, attn2)


def _k5_body(acc_ref, b2_ref,
             wih0, whh0, bih0, bhh0, wih1, whh1, bih1, bhh1,
             wih2, whh2, bih2, bhh2, out_ref):
    a = acc_ref[...]
    num = a[0, :N, :HID] + a[1, :N, :HID]
    den = a[0, :N, HID:HID + 1] + a[1, :N, HID:HID + 1]
    h = num / jnp.maximum(den, 1e-16) + b2_ref[...]
    feat = jnp.where(h > 0, h, jnp.exp(h) - 1.0)

    wihs = (wih0[...], wih1[...], wih2[...])
    whhs = (whh0[...], whh1[...], whh2[...])
    bihs = (bih0[...], bih1[...], bih2[...])
    bhhs = (bhh0[...], bhh1[...], bhh2[...])

    hs = [jnp.zeros((1, HID), jnp.float32) for _ in range(3)]
    cs = [jnp.zeros((1, HID), jnp.float32) for _ in range(3)]
    q_star = jnp.zeros((1, 2 * HID), jnp.float32)

    def dotT(u, w):
        return lax.dot_general(u, w, (((1,), (1,)), ((), ())),
                               preferred_element_type=jnp.float32)

    for _ in range(3):
        inp = q_star
        for l in range(3):
            gates = dotT(inp, wihs[l]) + bihs[l] + dotT(hs[l], whhs[l]) + bhhs[l]
            gi = gates[:, 0:HID]
            gf = gates[:, HID:2 * HID]
            gg = gates[:, 2 * HID:3 * HID]
            go = gates[:, 3 * HID:4 * HID]
            cnew = jax.nn.sigmoid(gf) * cs[l] + jax.nn.sigmoid(gi) * jnp.tanh(gg)
            hnew = jax.nn.sigmoid(go) * jnp.tanh(cnew)
            hs[l] = hnew
            cs[l] = cnew
            inp = hnew
        q = inp
        e = dotT(feat, q)
        m = jnp.max(e)
        z = jnp.exp(e - m)
        ssum = jnp.sum(z)
        r = lax.dot_general(z, feat, (((0,), (0,)), ((), ())),
                            preferred_element_type=jnp.float32) / ssum
        q_star = jnp.concatenate([q, r], axis=1)
    out_ref[...] = q_star


def _k5(acc2, bias2, lstm):
    return pl.pallas_call(
        _k5_body,
        out_shape=jax.ShapeDtypeStruct((1, 2 * HID), jnp.float32),
    )(acc2, bias2, *lstm)


def kernel(x, edge_index, W_src1, W_dst1, attn1, bias1, W_src2, W_dst2,
           attn2, bias2, W_ih0, W_hh0, b_ih0, b_hh0, W_ih1, W_hh1, b_ih1,
           b_hh1, W_ih2, W_hh2, b_ih2, b_hh2):
    src = edge_index[0]
    dst = edge_index[1]
    attn1_t = attn1.reshape(NC, 128)
    bias1_r = bias1.reshape(1, HEADS * HID)
    bias2_r = bias2.reshape(1, HID)

    fs_t, fd_t = _k1(x, W_src1, W_dst1)
    acc1, den1 = _k2(fs_t, fd_t, src, dst, attn1_t)
    den1_t = den1.transpose(3, 0, 2, 1).reshape(NR, NC * 2 * NS)
    ft2, = _k3(acc1, den1_t, bias1_r, W_src2, W_dst2)
    acc2 = _k4(ft2, src, dst, attn2)
    lstm = (W_ih0, W_hh0, b_ih0.reshape(1, -1), b_hh0.reshape(1, -1),
            W_ih1, W_hh1, b_ih1.reshape(1, -1), b_hh1.reshape(1, -1),
            W_ih2, W_hh2, b_ih2.reshape(1, -1), b_hh2.reshape(1, -1))
    return _k5(acc2, bias2_r, lstm)


# speedup vs baseline: 1.6203x; 1.0253x over previous
import functools

import jax
import jax.numpy as jnp
from jax import lax
from jax.experimental import pallas as pl
from jax.experimental.pallas import tpu as pltpu
from jax.experimental.pallas import tpu_sc as plsc

N = 10000
E = 160000
IN_FEAT = 128
HID = 64
HEADS = 4
NEG_SLOPE = 0.2

NC = 2
NS = 16
L = 16

NR = 10240
DUMP = N
RPT = NR // NS

ET1 = E // NS
EC1 = 48
NP1 = 104
TAILE = ET1 - NP1 * 2 * EC1
EC = 80
ET2 = E // (NC * NS)
NF2 = ET2 // EC
TAIL2 = ET2 - NF2 * EC

def _mesh():
    return plsc.VectorSubcoreMesh(core_axis_name="c", subcore_axis_name="s",
                                  num_cores=NC, num_subcores=NS)


def _k1_body(x_ref, ws_ref, wd_ref, fs_ref, fd_ref):
    x = x_ref[...]
    fs_ref[...] = jnp.dot(x, ws_ref[...], preferred_element_type=jnp.float32)
    fd_ref[...] = jnp.dot(x, wd_ref[...], preferred_element_type=jnp.float32)


def _k1(x, W_src1, W_dst1):
    nb = 10
    rb = N // nb
    return pl.pallas_call(
        _k1_body,
        grid=(NC, nb),
        in_specs=[
            pl.BlockSpec((rb, IN_FEAT), lambda c, j: (j, 0)),
            pl.BlockSpec((IN_FEAT, 128), lambda c, j: (0, c)),
            pl.BlockSpec((IN_FEAT, 128), lambda c, j: (0, c)),
        ],
        out_specs=[
            pl.BlockSpec((rb, 128), lambda c, j: (c * nb + j, 0)),
            pl.BlockSpec((rb, 128), lambda c, j: (c * nb + j, 0)),
        ],
        out_shape=[
            jax.ShapeDtypeStruct((NC * N, 128), jnp.float32),
            jax.ShapeDtypeStruct((NC * N, 128), jnp.float32),
        ],
    )(x, W_src1, W_dst1)


def _k2_body(fs_hbm, fd_hbm, src_hbm, dst_hbm, attn_hbm,
             out_hbm, outden_hbm,
             srcA, dofA, drwA, srcB, dofB, drwB,
             fsA, fdA, fsB, fdB, attn_v,
             den0_v, den1_v, acc_s,
             smA1, smA2, smB1, smB2, sgA1, sgA2, sgB1, sgB2, ssA, ssB):
    c = lax.axis_index("c")
    s = lax.axis_index("s")
    zf = jnp.zeros((L,), jnp.float32)
    lane = lax.iota(jnp.int32, L)

    def _zrow(r, carry):
        for k in range(128 // L):
            fsA[r, pl.ds(k * L, L)] = zf
        return carry
    lax.fori_loop(0, EC1, _zrow, 0)
    base = s * RPT
    for blk in range(RPT // EC1):
        pltpu.sync_copy(fsA, acc_s.at[pl.ds(base + blk * EC1, EC1)])
    remz = RPT - (RPT // EC1) * EC1
    pltpu.sync_copy(fsA.at[pl.ds(0, remz)],
                    acc_s.at[pl.ds(base + (RPT // EC1) * EC1, remz)])

    def _zden(i, carry):
        den0_v[pl.ds(i * L, L)] = zf
        den1_v[pl.ds(i * L, L)] = zf
        return carry
    lax.fori_loop(0, NR // L, _zden, 0)

    pltpu.sync_copy(attn_hbm.at[c], attn_v)
    a_vecs = [attn_v[pl.ds(k * L, L)] for k in range(8)]
    an_vecs = [a * NEG_SLOPE for a in a_vecs]
    masks = [lane == i for i in range(L)]

    coff = jnp.full((L,), c * N, jnp.int32)
    ebase = s * ET1

    plsc.subcore_barrier()

    def _offsets(src_ch, dof_ch, drw_ch):
        for g in range(EC1 // L):
            src_ch[pl.ds(g * L, L)] = src_ch[pl.ds(g * L, L)] + coff
            dof_ch[pl.ds(g * L, L)] = drw_ch[pl.ds(g * L, L)] + coff

    def _compute(fs_rows, fd_rows, drw_ch, ssc):
        def _group(g, carry):
            dvec = drw_ch[pl.ds(g * L, L)]
            for e16 in range(L):
                e = g * L + e16
                fsv = [fs_rows[e, pl.ds(k * L, L)] for k in range(8)]
                fdv = [fd_rows[e, pl.ds(k * L, L)] for k in range(8)]
                av = []
                for k in range(8):
                    ev = fsv[k] + fdv[k]
                    av.append(ev * jnp.where(ev > 0, a_vecs[k], an_vecs[k]))
                s0 = jnp.sum((av[0] + av[1]) + (av[2] + av[3]))
                s1 = jnp.sum((av[4] + av[5]) + (av[6] + av[7]))
                ex0 = jnp.exp(jnp.full((L,), s0, jnp.float32))
                ex1 = jnp.exp(jnp.full((L,), s1, jnp.float32))
                for k in range(4):
                    fs_rows[e, pl.ds(k * L, L)] = fsv[k] * ex0
                for k in range(4, 8):
                    fs_rows[e, pl.ds(k * L, L)] = fsv[k] * ex1
                plsc.addupdate_scatter(den0_v, [dvec], ex0, mask=masks[e16])
                plsc.addupdate_scatter(den1_v, [dvec], ex1, mask=masks[e16])
            return carry
        lax.fori_loop(0, EC1 // L, _group, 0)
        return pltpu.async_copy(fs_rows, acc_s.at[drw_ch], ssc, add=True)

    pltpu.async_copy(src_hbm.at[pl.ds(ebase, EC1)], srcA, smA1)
    pltpu.async_copy(dst_hbm.at[pl.ds(ebase, EC1)], drwA, smA2)
    pltpu.async_copy(src_hbm.at[pl.ds(ebase + EC1, EC1)], srcB, smB1)
    pltpu.async_copy(dst_hbm.at[pl.ds(ebase + EC1, EC1)], drwB, smB2)

    def _pair(p, carry):
        ebA = ebase + (2 * p) * EC1
        pltpu.make_async_copy(src_hbm.at[pl.ds(ebA, EC1)], srcA, smA1).wait()
        pltpu.make_async_copy(dst_hbm.at[pl.ds(ebA, EC1)], drwA, smA2).wait()
        _offsets(srcA, dofA, drwA)
        gA1 = pltpu.async_copy(fs_hbm.at[srcA], fsA, sgA1)
        gA2 = pltpu.async_copy(fd_hbm.at[dofA], fdA, sgA2)
        pltpu.make_async_copy(src_hbm.at[pl.ds(ebA, EC1)], srcB, smB1).wait()
        pltpu.make_async_copy(dst_hbm.at[pl.ds(ebA, EC1)], drwB, smB2).wait()
        _offsets(srcB, dofB, drwB)
        gB1 = pltpu.async_copy(fs_hbm.at[srcB], fsB, sgB1)
        gB2 = pltpu.async_copy(fd_hbm.at[dofB], fdB, sgB2)
        gA1.wait()
        gA2.wait()
        scA = _compute(fsA, fdA, drwA, ssA)
        gB1.wait()
        gB2.wait()
        scB = _compute(fsB, fdB, drwB, ssB)
        ebA2 = jnp.minimum(ebA + 2 * EC1, NS * ET1 - 2 * EC1)
        scA.wait()
        pltpu.async_copy(src_hbm.at[pl.ds(ebA2, EC1)], srcA, smA1)
        pltpu.async_copy(dst_hbm.at[pl.ds(ebA2, EC1)], drwA, smA2)
        scB.wait()
        pltpu.async_copy(src_hbm.at[pl.ds(ebA2 + EC1, EC1)], srcB, smB1)
        pltpu.async_copy(dst_hbm.at[pl.ds(ebA2 + EC1, EC1)], drwB, smB2)
        return carry
    lax.fori_loop(0, NP1, _pair, 0)
    pltpu.make_async_copy(src_hbm.at[pl.ds(0, EC1)], srcA, smA1).wait()
    pltpu.make_async_copy(dst_hbm.at[pl.ds(0, EC1)], drwA, smA2).wait()
    pltpu.make_async_copy(src_hbm.at[pl.ds(0, EC1)], srcB, smB1).wait()
    pltpu.make_async_copy(dst_hbm.at[pl.ds(0, EC1)], drwB, smB2).wait()

    pltpu.sync_copy(src_hbm.at[pl.ds(ebase + NP1 * 2 * EC1, TAILE)],
                    srcA.at[pl.ds(0, TAILE)])
    pltpu.sync_copy(dst_hbm.at[pl.ds(ebase + NP1 * 2 * EC1, TAILE)],
                    drwA.at[pl.ds(0, TAILE)])
    for t in range(TAILE // L, EC1 // L):
        srcA[pl.ds(t * L, L)] = jnp.full((L,), 0, jnp.int32)
        drwA[pl.ds(t * L, L)] = jnp.full((L,), DUMP, jnp.int32)
    _offsets(srcA, dofA, drwA)
    tA1 = pltpu.async_copy(fs_hbm.at[srcA], fsA, sgA1)
    tA2 = pltpu.async_copy(fd_hbm.at[dofA], fdA, sgA2)
    tA1.wait()
    tA2.wait()
    _compute(fsA, fdA, drwA, ssA).wait()

    pltpu.sync_copy(den0_v, outden_hbm.at[c, s, 0])
    pltpu.sync_copy(den1_v, outden_hbm.at[c, s, 1])

    plsc.subcore_barrier()
    pltpu.sync_copy(acc_s.at[pl.ds(base, RPT)], out_hbm.at[c, pl.ds(base, RPT)])


def _k2(fs_t, fd_t, src, dst, attn_t):
    f = functools.partial(
        pl.kernel,
        out_type=(
            jax.ShapeDtypeStruct((NC, NR, 128), jnp.float32),
            jax.ShapeDtypeStruct((NC, NS, 2, NR), jnp.float32),
        ),
        mesh=_mesh(),
        compiler_params=pltpu.CompilerParams(needs_layout_passes=False),
        scratch_types=[
            pltpu.VMEM((EC1,), jnp.int32),
            pltpu.VMEM((EC1,), jnp.int32),
            pltpu.VMEM((EC1,), jnp.int32),
            pltpu.VMEM((EC1,), jnp.int32),
            pltpu.VMEM((EC1,), jnp.int32),
            pltpu.VMEM((EC1,), jnp.int32),
            pltpu.VMEM((EC1, 128), jnp.float32),
            pltpu.VMEM((EC1, 128), jnp.float32),
            pltpu.VMEM((EC1, 128), jnp.float32),
            pltpu.VMEM((EC1, 128), jnp.float32),
            pltpu.VMEM((128,), jnp.float32),
            pltpu.VMEM((NR,), jnp.float32),
            pltpu.VMEM((NR,), jnp.float32),
            pltpu.VMEM_SHARED((NR, 128), jnp.float32),
            pltpu.SemaphoreType.DMA,
            pltpu.SemaphoreType.DMA,
            pltpu.SemaphoreType.DMA,
            pltpu.SemaphoreType.DMA,
            pltpu.SemaphoreType.DMA,
            pltpu.SemaphoreType.DMA,
            pltpu.SemaphoreType.DMA,
            pltpu.SemaphoreType.DMA,
            pltpu.SemaphoreType.DMA,
            pltpu.SemaphoreType.DMA,
        ],
    )(_k2_body)
    return f(fs_t, fd_t, src, dst, attn_t)


def _k3_body(acc_ref, den_ref, b_ref, ws_ref, wd_ref, ft_ref):
    a = acc_ref[...]
    d = den_ref[...]
    parts = []
    for c in range(NC):
        for k in range(2):
            num = a[c, :, k * 64:(k + 1) * 64]
            den = jnp.sum(d[:, (c * 2 + k) * NS:(c * 2 + k + 1) * NS],
                          axis=1, keepdims=True)
            parts.append(num / jnp.maximum(den, 1e-16))
    h = jnp.concatenate(parts, axis=1) + b_ref[...]
    h = jnp.where(h > 0, h, jnp.exp(h) - 1.0)
    ft_ref[...] = jnp.concatenate(
        [jnp.dot(h, ws_ref[...], preferred_element_type=jnp.float32),
         jnp.dot(h, wd_ref[...], preferred_element_type=jnp.float32)], axis=1)


def _k3(acc1, den1, bias1, W_src2, W_dst2):
    nb = 10
    rb = N // nb
    return pl.pallas_call(
        _k3_body,
        grid=(nb,),
        in_specs=[
            pl.BlockSpec((NC, rb, 128), lambda j: (0, j, 0)),
            pl.BlockSpec((rb, NC * 2 * NS), lambda j: (j, 0)),
            pl.BlockSpec((1, HEADS * HID), lambda j: (0, 0)),
            pl.BlockSpec((HEADS * HID, HID), lambda j: (0, 0)),
            pl.BlockSpec((HEADS * HID, HID), lambda j: (0, 0)),
        ],
        out_specs=[
            pl.BlockSpec((rb, 2 * HID), lambda j: (j, 0)),
        ],
        out_shape=[
            jax.ShapeDtypeStruct((N, 2 * HID), jnp.float32),
        ],
    )(acc1, den1, bias1, W_src2, W_dst2)


def _k4_body(ft_hbm, src_hbm, dst_hbm, attn_hbm, out_hbm,
             srcA, dstA, srcB, dstB, fsA, fdA, fsB, fdB, attn_v,
             acc_s, smA1, smA2, smB1, smB2, sgA1, sgA2, sgB1, sgB2, ssA, ssB):
    c = lax.axis_index("c")
    s = lax.axis_index("s")
    zf = jnp.zeros((L,), jnp.float32)
    lane = lax.iota(jnp.int32, L)

    def _zrow(r, carry):
        for k in range(128 // L):
            fsA[r, pl.ds(k * L, L)] = zf
        return carry
    lax.fori_loop(0, EC, _zrow, 0)
    base = s * RPT
    for blk in range(RPT // EC):
        pltpu.sync_copy(fsA, acc_s.at[pl.ds(base + blk * EC, EC)])

    pltpu.sync_copy(attn_hbm.at[0], attn_v)
    a_vecs = [attn_v[pl.ds(k * L, L)] for k in range(4)]
    an_vecs = [a * NEG_SLOPE for a in a_vecs]

    ebase = (c * NS + s) * ET2

    plsc.subcore_barrier()

    def _compute(fs_rows, fd_rows, dst_ch, ssc):
        def _edge(e, carry):
            fsv = [fs_rows[e, pl.ds(k * L, L)] for k in range(4)]
            fdv = [fd_rows[e, pl.ds((4 + k) * L, L)] for k in range(4)]
            av = []
            for k in range(4):
                ev = fsv[k] + fdv[k]
                av.append(ev * jnp.where(ev > 0, a_vecs[k], an_vecs[k]))
            s0 = jnp.sum((av[0] + av[1]) + (av[2] + av[3]))
            ex0 = jnp.exp(jnp.full((L,), s0, jnp.float32))
            for k in range(4):
                fs_rows[e, pl.ds(k * L, L)] = fsv[k] * ex0
            fs_rows[e, pl.ds(64, L)] = jnp.where(lane == 0, ex0, zf)
            for k in range(5, 8):
                fs_rows[e, pl.ds(k * L, L)] = zf
            return carry
        lax.fori_loop(0, EC, _edge, 0)
        return pltpu.async_copy(fs_rows, acc_s.at[dst_ch], ssc, add=True)

    pltpu.async_copy(src_hbm.at[pl.ds(ebase, EC)], srcA, smA1)
    pltpu.async_copy(dst_hbm.at[pl.ds(ebase, EC)], dstA, smA2)
    pltpu.async_copy(src_hbm.at[pl.ds(ebase + EC, EC)], srcB, smB1)
    pltpu.async_copy(dst_hbm.at[pl.ds(ebase + EC, EC)], dstB, smB2)

    def _pair(p, carry):
        ebA = ebase + (2 * p) * EC
        pltpu.make_async_copy(src_hbm.at[pl.ds(ebA, EC)], srcA, smA1).wait()
        pltpu.make_async_copy(dst_hbm.at[pl.ds(ebA, EC)], dstA, smA2).wait()
        gA1 = pltpu.async_copy(ft_hbm.at[srcA], fsA, sgA1)
        gA2 = pltpu.async_copy(ft_hbm.at[dstA], fdA, sgA2)
        pltpu.make_async_copy(src_hbm.at[pl.ds(ebA, EC)], srcB, smB1).wait()
        pltpu.make_async_copy(dst_hbm.at[pl.ds(ebA, EC)], dstB, smB2).wait()
        gB1 = pltpu.async_copy(ft_hbm.at[srcB], fsB, sgB1)
        gB2 = pltpu.async_copy(ft_hbm.at[dstB], fdB, sgB2)
        gA1.wait()
        gA2.wait()
        scA = _compute(fsA, fdA, dstA, ssA)
        gB1.wait()
        gB2.wait()
        scB = _compute(fsB, fdB, dstB, ssB)
        ebA2 = jnp.minimum(ebA + 2 * EC, E - 2 * EC)
        scA.wait()
        pltpu.async_copy(src_hbm.at[pl.ds(ebA2, EC)], srcA, smA1)
        pltpu.async_copy(dst_hbm.at[pl.ds(ebA2, EC)], dstA, smA2)
        scB.wait()
        pltpu.async_copy(src_hbm.at[pl.ds(ebA2 + EC, EC)], srcB, smB1)
        pltpu.async_copy(dst_hbm.at[pl.ds(ebA2 + EC, EC)], dstB, smB2)
        return carry
    lax.fori_loop(0, NF2 // 2, _pair, 0)
    pltpu.make_async_copy(src_hbm.at[pl.ds(0, EC)], srcA, smA1).wait()
    pltpu.make_async_copy(dst_hbm.at[pl.ds(0, EC)], dstA, smA2).wait()
    pltpu.make_async_copy(src_hbm.at[pl.ds(0, EC)], srcB, smB1).wait()
    pltpu.make_async_copy(dst_hbm.at[pl.ds(0, EC)], dstB, smB2).wait()

    pltpu.sync_copy(src_hbm.at[pl.ds(ebase + NF2 * EC, TAIL2)],
                    srcA.at[pl.ds(0, TAIL2)])
    pltpu.sync_copy(dst_hbm.at[pl.ds(ebase + NF2 * EC, TAIL2)],
                    dstA.at[pl.ds(0, TAIL2)])
    for t in range(EC // L):
        lo = t * L
        if TAIL2 >= lo + L:
            continue
        if TAIL2 > lo:
            keep = lane < (TAIL2 - lo)
            srcA[pl.ds(lo, L)] = jnp.where(keep, srcA[pl.ds(lo, L)], 0)
            dstA[pl.ds(lo, L)] = jnp.where(keep, dstA[pl.ds(lo, L)], DUMP)
        else:
            srcA[pl.ds(lo, L)] = jnp.full((L,), 0, jnp.int32)
            dstA[pl.ds(lo, L)] = jnp.full((L,), DUMP, jnp.int32)
    tA1 = pltpu.async_copy(ft_hbm.at[srcA], fsA, sgA1)
    tA2 = pltpu.async_copy(ft_hbm.at[dstA], fdA, sgA2)
    tA1.wait()
    tA2.wait()
    _compute(fsA, fdA, dstA, ssA).wait()

    plsc.subcore_barrier()
    pltpu.sync_copy(acc_s.at[pl.ds(base, RPT)], out_hbm.at[c, pl.ds(base, RPT)])


def _k4(ft2, src, dst, attn2):
    f = functools.partial(
        pl.kernel,
        out_type=jax.ShapeDtypeStruct((NC, NR, 128), jnp.float32),
        mesh=_mesh(),
        compiler_params=pltpu.CompilerParams(needs_layout_passes=False),
        scratch_types=[
            pltpu.VMEM((EC,), jnp.int32),
            pltpu.VMEM((EC,), jnp.int32),
            pltpu.VMEM((EC,), jnp.int32),
            pltpu.VMEM((EC,), jnp.int32),
            pltpu.VMEM((EC, 128), jnp.float32),
            pltpu.VMEM((EC, 128), jnp.float32),
            pltpu.VMEM((EC, 128), jnp.float32),
            pltpu.VMEM((EC, 128), jnp.float32),
            pltpu.VMEM((HID,), jnp.float32),
            pltpu.VMEM_SHARED((NR, 128), jnp.float32),
            pltpu.SemaphoreType.DMA,
            pltpu.SemaphoreType.DMA,
            pltpu.SemaphoreType.DMA,
            pltpu.SemaphoreType.DMA,
            pltpu.SemaphoreType.DMA,
            pltpu.SemaphoreType.DMA,
            pltpu.SemaphoreType.DMA,
            pltpu.SemaphoreType.DMA,
            pltpu.SemaphoreType.DMA,
            pltpu.SemaphoreType.DMA,
        ],
    )(_k4_body)
    return f(ft2, src, dst, attn2)


def _k5_body(acc_ref, b2_ref,
             wih0, whh0, bih0, bhh0, wih1, whh1, bih1, bhh1,
             wih2, whh2, bih2, bhh2, out_ref):
    a = acc_ref[...]
    num = a[0, :N, :HID] + a[1, :N, :HID]
    den = a[0, :N, HID:HID + 1] + a[1, :N, HID:HID + 1]
    h = num / jnp.maximum(den, 1e-16) + b2_ref[...]
    feat = jnp.where(h > 0, h, jnp.exp(h) - 1.0)

    wihs = (wih0[...], wih1[...], wih2[...])
    whhs = (whh0[...], whh1[...], whh2[...])
    bihs = (bih0[...], bih1[...], bih2[...])
    bhhs = (bhh0[...], bhh1[...], bhh2[...])

    hs = [jnp.zeros((1, HID), jnp.float32) for _ in range(3)]
    cs = [jnp.zeros((1, HID), jnp.float32) for _ in range(3)]
    q_star = jnp.zeros((1, 2 * HID), jnp.float32)

    def dotT(u, w):
        return lax.dot_general(u, w, (((1,), (1,)), ((), ())),
                               preferred_element_type=jnp.float32)

    for _ in range(3):
        inp = q_star
        for l in range(3):
            gates = dotT(inp, wihs[l]) + bihs[l] + dotT(hs[l], whhs[l]) + bhhs[l]
            gi = gates[:, 0:HID]
            gf = gates[:, HID:2 * HID]
            gg = gates[:, 2 * HID:3 * HID]
            go = gates[:, 3 * HID:4 * HID]
            cnew = jax.nn.sigmoid(gf) * cs[l] + jax.nn.sigmoid(gi) * jnp.tanh(gg)
            hnew = jax.nn.sigmoid(go) * jnp.tanh(cnew)
            hs[l] = hnew
            cs[l] = cnew
            inp = hnew
        q = inp
        e = dotT(feat, q)
        m = jnp.max(e)
        z = jnp.exp(e - m)
        ssum = jnp.sum(z)
        r = lax.dot_general(z, feat, (((0,), (0,)), ((), ())),
                            preferred_element_type=jnp.float32) / ssum
        q_star = jnp.concatenate([q, r], axis=1)
    out_ref[...] = q_star


def _k5(acc2, bias2, lstm):
    return pl.pallas_call(
        _k5_body,
        out_shape=jax.ShapeDtypeStruct((1, 2 * HID), jnp.float32),
    )(acc2, bias2, *lstm)


def kernel(x, edge_index, W_src1, W_dst1, attn1, bias1, W_src2, W_dst2,
           attn2, bias2, W_ih0, W_hh0, b_ih0, b_hh0, W_ih1, W_hh1, b_ih1,
           b_hh1, W_ih2, W_hh2, b_ih2, b_hh2):
    src = edge_index[0]
    dst = edge_index[1]
    attn1_t = attn1.reshape(NC, 128)
    bias1_r = bias1.reshape(1, HEADS * HID)
    bias2_r = bias2.reshape(1, HID)

    fs_t, fd_t = _k1(x, W_src1, W_dst1)
    acc1, den1 = _k2(fs_t, fd_t, src, dst, attn1_t)
    den1_t = den1.transpose(3, 0, 2, 1).reshape(NR, NC * 2 * NS)
    ft2, = _k3(acc1, den1_t, bias1_r, W_src2, W_dst2)
    acc2 = _k4(ft2, src, dst, attn2)
    lstm = (W_ih0, W_hh0, b_ih0.reshape(1, -1), b_hh0.reshape(1, -1),
            W_ih1, W_hh1, b_ih1.reshape(1, -1), b_hh1.reshape(1, -1),
            W_ih2, W_hh2, b_ih2.reshape(1, -1), b_hh2.reshape(1, -1))
    return _k5(acc2, bias2_r, lstm)
